# Initial kernel scaffold; baseline (speedup 1.0000x reference)
import jax
import jax.numpy as jnp
from jax.experimental import pallas as pl


def kernel(x, edge_index, batch, W0, b0, W1, b1, W2, b2, W3, b3, Wout, bout):
    raise NotImplementedError("write your pallas kernel here")


# speedup vs baseline: 13.9305x; 13.9305x over previous
import jax
import jax.numpy as jnp
import numpy as np
from jax import lax
from jax.experimental import pallas as pl
from jax.experimental.pallas import tpu as pltpu
from jax.experimental.pallas import tpu_sc as plsc

N = 50000
E = 800000
B = 16
DIN = 128
DH = 64
DOUT = 64
HALF = 32

NC = 2
NS = 16
NT = NC * NS

WIN = 128
EP = 802816
PAD = EP - E
W_DEG = EP // (NT * WIN)
W_SP = EP // (NS * WIN)
KW = 8

ROWS_T = 3136
N_ACC = NS * ROWS_T
ZCH = 24

R_TC = 1000
G_TC = N // R_TC


def _fill_zeros_1d(ref, n):
    zero = jnp.zeros((16,), jnp.float32)
    for i in range(n // 16):
        ref[pl.ds(i * 16, 16)] = zero


def _sc_deg_body(dst_hbm, deg_out, acc, idx_v, ones_v, zeros_v):
    c = lax.axis_index("c")
    s = lax.axis_index("s")
    g = c * NS + s
    one = jnp.ones((16,), jnp.float32)
    for i in range(WIN // 16):
        ones_v[pl.ds(i * 16, 16)] = one
    _fill_zeros_1d(zeros_v, ROWS_T)
    pltpu.sync_copy(dst_hbm.at[g], idx_v)
    pltpu.sync_copy(zeros_v, acc.at[pl.ds(s * ROWS_T, ROWS_T)])
    plsc.subcore_barrier()

    def w_body(w, carry):
        pltpu.sync_copy(ones_v, acc.at[idx_v.at[w]], add=True)
        return carry

    lax.fori_loop(0, W_DEG, w_body, 0)
    plsc.subcore_barrier()
    pltpu.sync_copy(acc.at[pl.ds(s * ROWS_T, ROWS_T)], zeros_v)
    pltpu.sync_copy(zeros_v, deg_out.at[pl.ds(c * N_ACC + s * ROWS_T, ROWS_T)])


def _sc_deg(dst32):
    mesh = plsc.VectorSubcoreMesh(core_axis_name="c", subcore_axis_name="s")
    return pl.kernel(
        _sc_deg_body,
        out_type=jax.ShapeDtypeStruct((NC * N_ACC,), jnp.float32),
        mesh=mesh,
        scratch_types=[
            pltpu.VMEM_SHARED((N_ACC,), jnp.float32),
            pltpu.VMEM((W_DEG, WIN), jnp.int32),
            pltpu.VMEM((WIN,), jnp.float32),
            pltpu.VMEM((ROWS_T,), jnp.float32),
        ],
    )(dst32)


def _sc_spmm_body(table_hbm, src_hbm, dst_hbm, s_out, acc, sidx_v, didx_v,
                  rows_v):
    c = lax.axis_index("c")
    s = lax.axis_index("s")
    g = c * NS + s
    zero = jnp.zeros((16,), jnp.float32)
    for i in range(WIN):
        rows_v[i, pl.ds(0, 16)] = zero
        rows_v[i, pl.ds(16, 16)] = zero
    for k in range(ZCH):
        pltpu.sync_copy(rows_v, acc.at[pl.ds(s * ROWS_T + k * WIN, WIN)])
    pltpu.sync_copy(rows_v.at[pl.ds(0, ROWS_T - ZCH * WIN)],
                    acc.at[pl.ds(s * ROWS_T + ZCH * WIN, ROWS_T - ZCH * WIN)])
    plsc.subcore_barrier()

    def blk_body(blk, carry):
        pltpu.sync_copy(src_hbm.at[g, pl.ds(blk * KW, KW)], sidx_v)
        pltpu.sync_copy(dst_hbm.at[g, pl.ds(blk * KW, KW)], didx_v)

        def w_body(w, carry2):
            pltpu.sync_copy(table_hbm.at[sidx_v.at[w]], rows_v)
            pltpu.sync_copy(rows_v, acc.at[didx_v.at[w]], add=True)
            return carry2

        return lax.fori_loop(0, KW, w_body, carry)

    lax.fori_loop(0, W_SP // KW, blk_body, 0)
    plsc.subcore_barrier()
    base = c * N_ACC + s * ROWS_T
    for k in range(ZCH):
        pltpu.sync_copy(acc.at[pl.ds(s * ROWS_T + k * WIN, WIN)], rows_v)
        pltpu.sync_copy(rows_v, s_out.at[pl.ds(base + k * WIN, WIN)])
    tail = ROWS_T - ZCH * WIN
    pltpu.sync_copy(acc.at[pl.ds(s * ROWS_T + ZCH * WIN, tail)],
                    rows_v.at[pl.ds(0, tail)])
    pltpu.sync_copy(rows_v.at[pl.ds(0, tail)],
                    s_out.at[pl.ds(base + ZCH * WIN, tail)])


def _sc_spmm(table, src32, dst32):
    mesh = plsc.VectorSubcoreMesh(core_axis_name="c", subcore_axis_name="s")
    return pl.kernel(
        _sc_spmm_body,
        out_type=jax.ShapeDtypeStruct((NC * N_ACC, HALF), jnp.float32),
        mesh=mesh,
        compiler_params=pltpu.CompilerParams(use_tc_tiling_on_sc=False),
        scratch_types=[
            pltpu.VMEM_SHARED((N_ACC, HALF), jnp.float32),
            pltpu.VMEM((KW, WIN), jnp.int32),
            pltpu.VMEM((KW, WIN), jnp.int32),
            pltpu.VMEM((WIN, HALF), jnp.float32),
        ],
    )(table, src32, dst32)


def _tc_prep_body(p0_ref, p1_ref, x_ref, w_ref, dinv_ref, t_ref):
    deg = 1.0 + p0_ref[...] + p1_ref[...]
    dv = lax.rsqrt(deg)
    t = dv * jnp.dot(x_ref[...], w_ref[...], preferred_element_type=jnp.float32)
    dinv_ref[...] = dv
    t_ref[0] = t[:, :HALF]
    t_ref[1] = t[:, HALF:]


def _tc_prep(p0, p1, x, w0):
    return pl.pallas_call(
        _tc_prep_body,
        grid=(G_TC,),
        in_specs=[
            pl.BlockSpec((R_TC, 1), lambda i: (i, 0)),
            pl.BlockSpec((R_TC, 1), lambda i: (i, 0)),
            pl.BlockSpec((R_TC, DIN), lambda i: (i, 0)),
            pl.BlockSpec((DIN, DH), lambda i: (0, 0)),
        ],
        out_specs=[
            pl.BlockSpec((R_TC, 1), lambda i: (i, 0)),
            pl.BlockSpec((NC, R_TC, HALF), lambda i: (0, i, 0)),
        ],
        out_shape=[
            jax.ShapeDtypeStruct((N, 1), jnp.float32),
            jax.ShapeDtypeStruct((NC, N, HALF), jnp.float32),
        ],
    )(p0, p1, x, w0)


def _tc_layer_body(s_ref, tp_ref, dinv_ref, w_ref, b_ref, t_ref):
    u = jnp.concatenate([s_ref[0] + tp_ref[0], s_ref[1] + tp_ref[1]], axis=1)
    dv = dinv_ref[...]
    h = jnp.maximum(dv * u + b_ref[...], 0.0)
    t = dv * jnp.dot(h, w_ref[...], preferred_element_type=jnp.float32)
    t_ref[0] = t[:, :HALF]
    t_ref[1] = t[:, HALF:]


def _tc_layer(s, tp, dinv, w, b):
    return pl.pallas_call(
        _tc_layer_body,
        grid=(G_TC,),
        in_specs=[
            pl.BlockSpec((NC, R_TC, HALF), lambda i: (0, i, 0)),
            pl.BlockSpec((NC, R_TC, HALF), lambda i: (0, i, 0)),
            pl.BlockSpec((R_TC, 1), lambda i: (i, 0)),
            pl.BlockSpec((DH, DH), lambda i: (0, 0)),
            pl.BlockSpec((1, DH), lambda i: (0, 0)),
        ],
        out_specs=pl.BlockSpec((NC, R_TC, HALF), lambda i: (0, i, 0)),
        out_shape=jax.ShapeDtypeStruct((NC, N, HALF), jnp.float32),
    )(s, tp, dinv, w, b)


def _tc_final_body(s_ref, tp_ref, dinv_ref, b_ref, batch_ref, wout_ref,
                   bout_ref, o_ref, pool_acc):
    i = pl.program_id(0)
    u = jnp.concatenate([s_ref[0] + tp_ref[0], s_ref[1] + tp_ref[1]], axis=1)
    dv = dinv_ref[...]
    h = jnp.maximum(dv * u + b_ref[...], 0.0)
    hh = jnp.concatenate([h, jnp.ones((R_TC, 1), jnp.float32)], axis=1)
    oh = (batch_ref[...] == lax.broadcasted_iota(jnp.int32, (1, B), 1))
    oh = oh.astype(jnp.float32)
    pp = lax.dot_general(oh, hh, (((0,), (0,)), ((), ())),
                         preferred_element_type=jnp.float32)

    @pl.when(i == 0)
    def _():
        pool_acc[...] = pp

    @pl.when(i > 0)
    def _():
        pool_acc[...] += pp

    @pl.when(i == G_TC - 1)
    def _():
        cnt = pool_acc[:, DH:DH + 1]
        scl = 1.0 / (jnp.maximum(cnt, 1.0) * jnp.sqrt(cnt + 1e-6))
        o_ref[...] = jnp.dot(pool_acc[:, :DH] * scl, wout_ref[...],
                             preferred_element_type=jnp.float32) + bout_ref[...]


def _tc_final(s, tp, dinv, b, batch2, wout, bout):
    return pl.pallas_call(
        _tc_final_body,
        grid=(G_TC,),
        in_specs=[
            pl.BlockSpec((NC, R_TC, HALF), lambda i: (0, i, 0)),
            pl.BlockSpec((NC, R_TC, HALF), lambda i: (0, i, 0)),
            pl.BlockSpec((R_TC, 1), lambda i: (i, 0)),
            pl.BlockSpec((1, DH), lambda i: (0, 0)),
            pl.BlockSpec((R_TC, 1), lambda i: (i, 0)),
            pl.BlockSpec((DH, DOUT), lambda i: (0, 0)),
            pl.BlockSpec((1, DOUT), lambda i: (0, 0)),
        ],
        out_specs=pl.BlockSpec((B, DOUT), lambda i: (0, 0)),
        out_shape=jax.ShapeDtypeStruct((B, DOUT), jnp.float32),
        scratch_shapes=[pltpu.VMEM((B, DH + 1), jnp.float32)],
    )(s, tp, dinv, b, batch2, wout, bout)


def kernel(x, edge_index, batch, W0, b0, W1, b1, W2, b2, W3, b3, Wout, bout):
    src = edge_index[0]
    dst = edge_index[1]
    pad_i = jnp.arange(PAD, dtype=jnp.int32)
    src_p = jnp.concatenate([src, pad_i % np.int32(N)])
    dst_p = jnp.concatenate([dst, N + (pad_i % np.int32(16))])

    dst_deg = dst_p.reshape(NT, W_DEG, WIN)
    src_t = src_p.reshape(1, NS, W_SP, WIN)
    src_sp = jnp.concatenate([src_t, src_t + N], axis=0).reshape(NT, W_SP, WIN)
    dst_sp = jnp.broadcast_to(dst_p.reshape(1, NS, W_SP, WIN),
                              (NC, NS, W_SP, WIN)).reshape(NT, W_SP, WIN)

    deg_raw = _sc_deg(dst_deg)
    p0 = deg_raw[:N].reshape(N, 1)
    p1 = deg_raw[N_ACC:N_ACC + N].reshape(N, 1)

    dinv, t = _tc_prep(p0, p1, x, W0)
    for (w, b) in ((W1, b0), (W2, b1), (W3, b2)):
        s = _sc_spmm(t.reshape(NC * N, HALF), src_sp, dst_sp)
        t = _tc_layer(s.reshape(NC, N_ACC, HALF), t, dinv, w, b.reshape(1, DH))
    s = _sc_spmm(t.reshape(NC * N, HALF), src_sp, dst_sp)
    return _tc_final(s.reshape(NC, N_ACC, HALF), t, dinv, b3.reshape(1, DH),
                     batch.reshape(N, 1).astype(jnp.int32), Wout,
                     bout.reshape(1, DOUT))


# speedup vs baseline: 16.6594x; 1.1959x over previous
import jax
import jax.numpy as jnp
import numpy as np
from jax import lax
from jax.experimental import pallas as pl
from jax.experimental.pallas import tpu as pltpu
from jax.experimental.pallas import tpu_sc as plsc

N = 50000
E = 800000
B = 16
DIN = 128
DH = 64
DOUT = 64
HALF = 32

NC = 2
NS = 16
NT = NC * NS

WIN = 128
EP = 802816
PAD = EP - E
W_DEG = EP // (NT * WIN)
W_SP = EP // (NS * WIN)
KW = 8

ROWS_T = 3136
N_ACC = NS * ROWS_T
ZCH = 24

R_TC = 1000
G_TC = N // R_TC


def _fill_zeros_1d(ref, n):
    zero = jnp.zeros((16,), jnp.float32)
    for i in range(n // 16):
        ref[pl.ds(i * 16, 16)] = zero


def _sc_deg_body(dst_hbm, deg_out, acc, idx_v, ones_v, zeros_v):
    c = lax.axis_index("c")
    s = lax.axis_index("s")
    g = c * NS + s
    one = jnp.ones((16,), jnp.float32)
    for i in range(WIN // 16):
        ones_v[pl.ds(i * 16, 16)] = one
    _fill_zeros_1d(zeros_v, ROWS_T)
    pltpu.sync_copy(dst_hbm.at[g], idx_v)
    pltpu.sync_copy(zeros_v, acc.at[pl.ds(s * ROWS_T, ROWS_T)])
    plsc.subcore_barrier()

    def w_body(w, carry):
        pltpu.sync_copy(ones_v, acc.at[idx_v.at[w]], add=True)
        return carry

    lax.fori_loop(0, W_DEG, w_body, 0)
    plsc.subcore_barrier()
    pltpu.sync_copy(acc.at[pl.ds(s * ROWS_T, ROWS_T)], zeros_v)
    pltpu.sync_copy(zeros_v, deg_out.at[pl.ds(c * N_ACC + s * ROWS_T, ROWS_T)])


def _sc_deg(dst32):
    mesh = plsc.VectorSubcoreMesh(core_axis_name="c", subcore_axis_name="s")
    return pl.kernel(
        _sc_deg_body,
        out_type=jax.ShapeDtypeStruct((NC * N_ACC,), jnp.float32),
        mesh=mesh,
        scratch_types=[
            pltpu.VMEM_SHARED((N_ACC,), jnp.float32),
            pltpu.VMEM((W_DEG, WIN), jnp.int32),
            pltpu.VMEM((WIN,), jnp.float32),
            pltpu.VMEM((ROWS_T,), jnp.float32),
        ],
    )(dst32)


def _sc_spmm_body(table_hbm, src_hbm, dst_hbm, s_out, acc, sidx_v, didx_v,
                  rows_v, gsem0, gsem1):
    c = lax.axis_index("c")
    s = lax.axis_index("s")
    g = c * NS + s
    gsems = (gsem0, gsem1)
    zero = jnp.zeros((16,), jnp.float32)
    for i in range(WIN):
        rows_v[0, i, pl.ds(0, 16)] = zero
        rows_v[0, i, pl.ds(16, 16)] = zero
    for k in range(ZCH):
        pltpu.sync_copy(rows_v.at[0], acc.at[pl.ds(s * ROWS_T + k * WIN, WIN)])
    pltpu.sync_copy(rows_v.at[0, pl.ds(0, ROWS_T - ZCH * WIN)],
                    acc.at[pl.ds(s * ROWS_T + ZCH * WIN, ROWS_T - ZCH * WIN)])
    plsc.subcore_barrier()

    n_blk = W_SP // KW
    pltpu.sync_copy(src_hbm.at[g, pl.ds(0, KW)], sidx_v.at[0])
    pltpu.sync_copy(dst_hbm.at[g, pl.ds(0, KW)], didx_v.at[0])
    pltpu.async_copy(table_hbm.at[sidx_v.at[0, 0]], rows_v.at[0], gsem0)

    def blk_body(blk, carry):
        rb = blk % 2

        @pl.when(blk + 1 < n_blk)
        def _():
            pltpu.sync_copy(src_hbm.at[g, pl.ds((blk + 1) * KW, KW)],
                            sidx_v.at[1 - rb])
            pltpu.sync_copy(dst_hbm.at[g, pl.ds((blk + 1) * KW, KW)],
                            didx_v.at[1 - rb])

        for j in range(KW):
            b = j % 2
            pltpu.make_async_copy(table_hbm.at[sidx_v.at[rb, j]],
                                  rows_v.at[b], gsems[b]).wait()
            if j < KW - 1:
                pltpu.async_copy(table_hbm.at[sidx_v.at[rb, j + 1]],
                                 rows_v.at[1 - b], gsems[1 - b])
            else:
                @pl.when(blk + 1 < n_blk)
                def _():
                    pltpu.async_copy(table_hbm.at[sidx_v.at[1 - rb, 0]],
                                     rows_v.at[1 - b], gsems[1 - b])
            pltpu.sync_copy(rows_v.at[b], acc.at[didx_v.at[rb, j]], add=True)
        return carry

    lax.fori_loop(0, n_blk, blk_body, 0)
    plsc.subcore_barrier()
    base = c * N_ACC + s * ROWS_T
    for k in range(ZCH):
        pltpu.sync_copy(acc.at[pl.ds(s * ROWS_T + k * WIN, WIN)], rows_v.at[0])
        pltpu.sync_copy(rows_v.at[0], s_out.at[pl.ds(base + k * WIN, WIN)])
    tail = ROWS_T - ZCH * WIN
    pltpu.sync_copy(acc.at[pl.ds(s * ROWS_T + ZCH * WIN, tail)],
                    rows_v.at[0, pl.ds(0, tail)])
    pltpu.sync_copy(rows_v.at[0, pl.ds(0, tail)],
                    s_out.at[pl.ds(base + ZCH * WIN, tail)])


def _sc_spmm(table, src32, dst32):
    mesh = plsc.VectorSubcoreMesh(core_axis_name="c", subcore_axis_name="s")
    return pl.kernel(
        _sc_spmm_body,
        out_type=jax.ShapeDtypeStruct((NC * N_ACC, HALF), jnp.float32),
        mesh=mesh,
        compiler_params=pltpu.CompilerParams(use_tc_tiling_on_sc=False),
        scratch_types=[
            pltpu.VMEM_SHARED((N_ACC, HALF), jnp.float32),
            pltpu.VMEM((2, KW, WIN), jnp.int32),
            pltpu.VMEM((2, KW, WIN), jnp.int32),
            pltpu.VMEM((2, WIN, HALF), jnp.float32),
            pltpu.SemaphoreType.DMA,
            pltpu.SemaphoreType.DMA,
        ],
    )(table, src32, dst32)


def _tc_prep_body(p0_ref, p1_ref, x_ref, w_ref, dinv_ref, t_ref):
    deg = 1.0 + p0_ref[...] + p1_ref[...]
    dv = lax.rsqrt(deg)
    t = dv * jnp.dot(x_ref[...], w_ref[...], preferred_element_type=jnp.float32)
    dinv_ref[...] = dv
    t_ref[0] = t[:, :HALF]
    t_ref[1] = t[:, HALF:]


def _tc_prep(p0, p1, x, w0):
    return pl.pallas_call(
        _tc_prep_body,
        grid=(G_TC,),
        in_specs=[
            pl.BlockSpec((R_TC, 1), lambda i: (i, 0)),
            pl.BlockSpec((R_TC, 1), lambda i: (i, 0)),
            pl.BlockSpec((R_TC, DIN), lambda i: (i, 0)),
            pl.BlockSpec((DIN, DH), lambda i: (0, 0)),
        ],
        out_specs=[
            pl.BlockSpec((R_TC, 1), lambda i: (i, 0)),
            pl.BlockSpec((NC, R_TC, HALF), lambda i: (0, i, 0)),
        ],
        out_shape=[
            jax.ShapeDtypeStruct((N, 1), jnp.float32),
            jax.ShapeDtypeStruct((NC, N, HALF), jnp.float32),
        ],
    )(p0, p1, x, w0)


def _tc_layer_body(s_ref, tp_ref, dinv_ref, w_ref, b_ref, t_ref):
    u = jnp.concatenate([s_ref[0] + tp_ref[0], s_ref[1] + tp_ref[1]], axis=1)
    dv = dinv_ref[...]
    h = jnp.maximum(dv * u + b_ref[...], 0.0)
    t = dv * jnp.dot(h, w_ref[...], preferred_element_type=jnp.float32)
    t_ref[0] = t[:, :HALF]
    t_ref[1] = t[:, HALF:]


def _tc_layer(s, tp, dinv, w, b):
    return pl.pallas_call(
        _tc_layer_body,
        grid=(G_TC,),
        in_specs=[
            pl.BlockSpec((NC, R_TC, HALF), lambda i: (0, i, 0)),
            pl.BlockSpec((NC, R_TC, HALF), lambda i: (0, i, 0)),
            pl.BlockSpec((R_TC, 1), lambda i: (i, 0)),
            pl.BlockSpec((DH, DH), lambda i: (0, 0)),
            pl.BlockSpec((1, DH), lambda i: (0, 0)),
        ],
        out_specs=pl.BlockSpec((NC, R_TC, HALF), lambda i: (0, i, 0)),
        out_shape=jax.ShapeDtypeStruct((NC, N, HALF), jnp.float32),
    )(s, tp, dinv, w, b)


def _tc_final_body(s_ref, tp_ref, dinv_ref, b_ref, batch_ref, wout_ref,
                   bout_ref, o_ref, pool_acc):
    i = pl.program_id(0)
    u = jnp.concatenate([s_ref[0] + tp_ref[0], s_ref[1] + tp_ref[1]], axis=1)
    dv = dinv_ref[...]
    h = jnp.maximum(dv * u + b_ref[...], 0.0)
    hh = jnp.concatenate([h, jnp.ones((R_TC, 1), jnp.float32)], axis=1)
    oh = (batch_ref[...] == lax.broadcasted_iota(jnp.int32, (1, B), 1))
    oh = oh.astype(jnp.float32)
    pp = lax.dot_general(oh, hh, (((0,), (0,)), ((), ())),
                         preferred_element_type=jnp.float32)

    @pl.when(i == 0)
    def _():
        pool_acc[...] = pp

    @pl.when(i > 0)
    def _():
        pool_acc[...] += pp

    @pl.when(i == G_TC - 1)
    def _():
        cnt = pool_acc[:, DH:DH + 1]
        scl = 1.0 / (jnp.maximum(cnt, 1.0) * jnp.sqrt(cnt + 1e-6))
        o_ref[...] = jnp.dot(pool_acc[:, :DH] * scl, wout_ref[...],
                             preferred_element_type=jnp.float32) + bout_ref[...]


def _tc_final(s, tp, dinv, b, batch2, wout, bout):
    return pl.pallas_call(
        _tc_final_body,
        grid=(G_TC,),
        in_specs=[
            pl.BlockSpec((NC, R_TC, HALF), lambda i: (0, i, 0)),
            pl.BlockSpec((NC, R_TC, HALF), lambda i: (0, i, 0)),
            pl.BlockSpec((R_TC, 1), lambda i: (i, 0)),
            pl.BlockSpec((1, DH), lambda i: (0, 0)),
            pl.BlockSpec((R_TC, 1), lambda i: (i, 0)),
            pl.BlockSpec((DH, DOUT), lambda i: (0, 0)),
            pl.BlockSpec((1, DOUT), lambda i: (0, 0)),
        ],
        out_specs=pl.BlockSpec((B, DOUT), lambda i: (0, 0)),
        out_shape=jax.ShapeDtypeStruct((B, DOUT), jnp.float32),
        scratch_shapes=[pltpu.VMEM((B, DH + 1), jnp.float32)],
    )(s, tp, dinv, b, batch2, wout, bout)


def kernel(x, edge_index, batch, W0, b0, W1, b1, W2, b2, W3, b3, Wout, bout):
    src = edge_index[0]
    dst = edge_index[1]
    pad_i = jnp.arange(PAD, dtype=jnp.int32)
    src_p = jnp.concatenate([src, pad_i % np.int32(N)])
    dst_p = jnp.concatenate([dst, N + (pad_i % np.int32(16))])

    dst_deg = dst_p.reshape(NT, W_DEG, WIN)
    src_t = src_p.reshape(1, NS, W_SP, WIN)
    src_sp = jnp.concatenate([src_t, src_t + N], axis=0).reshape(NT, W_SP, WIN)
    dst_sp = jnp.broadcast_to(dst_p.reshape(1, NS, W_SP, WIN),
                              (NC, NS, W_SP, WIN)).reshape(NT, W_SP, WIN)

    deg_raw = _sc_deg(dst_deg)
    p0 = deg_raw[:N].reshape(N, 1)
    p1 = deg_raw[N_ACC:N_ACC + N].reshape(N, 1)

    dinv, t = _tc_prep(p0, p1, x, W0)
    for (w, b) in ((W1, b0), (W2, b1), (W3, b2)):
        s = _sc_spmm(t.reshape(NC * N, HALF), src_sp, dst_sp)
        t = _tc_layer(s.reshape(NC, N_ACC, HALF), t, dinv, w, b.reshape(1, DH))
    s = _sc_spmm(t.reshape(NC * N, HALF), src_sp, dst_sp)
    return _tc_final(s.reshape(NC, N_ACC, HALF), t, dinv, b3.reshape(1, DH),
                     batch.reshape(N, 1).astype(jnp.int32), Wout,
                     bout.reshape(1, DOUT))


# speedup vs baseline: 21.0026x; 1.2607x over previous
import jax
import jax.numpy as jnp
import numpy as np
from jax import lax
from jax.experimental import pallas as pl
from jax.experimental.pallas import tpu as pltpu
from jax.experimental.pallas import tpu_sc as plsc

N = 50000
E = 800000
B = 16
DIN = 128
DH = 64
DOUT = 64
HALF = 32

NC = 2
NS = 16
NT = NC * NS

WIN = 128
EP = 802816
PAD = EP - E
W_DEG = EP // (NT * WIN)
W_SP = EP // (NS * WIN)
KW = 8

ROWS_T = 3136
N_ACC = NS * ROWS_T
ZCH = 24

R_TC = 1000
G_TC = N // R_TC


def _fill_zeros_1d(ref, n):
    zero = jnp.zeros((16,), jnp.float32)
    for i in range(n // 16):
        ref[pl.ds(i * 16, 16)] = zero


def _sc_deg_body(dst_hbm, deg_out, acc, idx_v, ones_v, zeros_v):
    c = lax.axis_index("c")
    s = lax.axis_index("s")
    g = c * NS + s
    one = jnp.ones((16,), jnp.float32)
    for i in range(WIN // 16):
        ones_v[pl.ds(i * 16, 16)] = one
    _fill_zeros_1d(zeros_v, ROWS_T)
    pltpu.sync_copy(dst_hbm.at[g], idx_v)
    pltpu.sync_copy(zeros_v, acc.at[pl.ds(s * ROWS_T, ROWS_T)])
    plsc.subcore_barrier()

    def w_body(w, carry):
        pltpu.sync_copy(ones_v, acc.at[idx_v.at[w]], add=True)
        return carry

    lax.fori_loop(0, W_DEG, w_body, 0)
    plsc.subcore_barrier()
    pltpu.sync_copy(acc.at[pl.ds(s * ROWS_T, ROWS_T)], zeros_v)
    pltpu.sync_copy(zeros_v, deg_out.at[pl.ds(c * N_ACC + s * ROWS_T, ROWS_T)])


def _sc_deg(dst32):
    mesh = plsc.VectorSubcoreMesh(core_axis_name="c", subcore_axis_name="s")
    return pl.kernel(
        _sc_deg_body,
        out_type=jax.ShapeDtypeStruct((NC * N_ACC,), jnp.float32),
        mesh=mesh,
        scratch_types=[
            pltpu.VMEM_SHARED((N_ACC,), jnp.float32),
            pltpu.VMEM((W_DEG, WIN), jnp.int32),
            pltpu.VMEM((WIN,), jnp.float32),
            pltpu.VMEM((ROWS_T,), jnp.float32),
        ],
    )(dst32)


def _sc_spmm_body(table_hbm, src_hbm, dst_hbm, s_out, acc, sidx_v, didx_v,
                  rows_v, g0, g1, g2, g3, s0, s1, s2, s3):
    c = lax.axis_index("c")
    s = lax.axis_index("s")
    g = c * NS + s
    gsems = (g0, g1, g2, g3)
    ssems = (s0, s1, s2, s3)
    zero = jnp.zeros((16,), jnp.float32)
    for i in range(WIN):
        rows_v[0, i, pl.ds(0, 16)] = zero
        rows_v[0, i, pl.ds(16, 16)] = zero
    for k in range(ZCH):
        pltpu.sync_copy(rows_v.at[0], acc.at[pl.ds(s * ROWS_T + k * WIN, WIN)])
    pltpu.sync_copy(rows_v.at[0, pl.ds(0, ROWS_T - ZCH * WIN)],
                    acc.at[pl.ds(s * ROWS_T + ZCH * WIN, ROWS_T - ZCH * WIN)])
    plsc.subcore_barrier()

    n_blk = W_SP // KW
    pltpu.sync_copy(src_hbm.at[g, pl.ds(0, KW)], sidx_v.at[0])
    pltpu.sync_copy(dst_hbm.at[g, pl.ds(0, KW)], didx_v.at[0])
    pltpu.async_copy(table_hbm.at[sidx_v.at[0, 0]], rows_v.at[0], gsems[0])
    pltpu.async_copy(table_hbm.at[sidx_v.at[0, 1]], rows_v.at[1], gsems[1])

    def blk_body(blk, carry):
        rb = blk % 2
        for j in range(KW):
            b = j % 4
            bp2 = (j + 2) % 4
            pltpu.make_async_copy(table_hbm.at[sidx_v.at[rb, j]],
                                  rows_v.at[b], gsems[b]).wait()
            if j >= 2:
                pltpu.make_async_copy(rows_v.at[bp2],
                                      acc.at[didx_v.at[rb, j - 2]],
                                      ssems[bp2]).wait()
            else:
                @pl.when(blk > 0)
                def _():
                    pltpu.make_async_copy(rows_v.at[bp2],
                                          acc.at[didx_v.at[rb, j]],
                                          ssems[bp2]).wait()
            if j == 2:
                @pl.when(blk + 1 < n_blk)
                def _():
                    pltpu.sync_copy(src_hbm.at[g, pl.ds((blk + 1) * KW, KW)],
                                    sidx_v.at[1 - rb])
                    pltpu.sync_copy(dst_hbm.at[g, pl.ds((blk + 1) * KW, KW)],
                                    didx_v.at[1 - rb])
            if j < KW - 2:
                pltpu.async_copy(table_hbm.at[sidx_v.at[rb, j + 2]],
                                 rows_v.at[bp2], gsems[bp2])
            else:
                @pl.when(blk + 1 < n_blk)
                def _():
                    pltpu.async_copy(table_hbm.at[sidx_v.at[1 - rb, j - (KW - 2)]],
                                     rows_v.at[bp2], gsems[bp2])
            pltpu.async_copy(rows_v.at[b], acc.at[didx_v.at[rb, j]], ssems[b],
                             add=True)
        return carry

    lax.fori_loop(0, n_blk, blk_body, 0)
    rbl = (n_blk - 1) % 2
    pltpu.make_async_copy(rows_v.at[2], acc.at[didx_v.at[rbl, KW - 2]],
                          ssems[2]).wait()
    pltpu.make_async_copy(rows_v.at[3], acc.at[didx_v.at[rbl, KW - 1]],
                          ssems[3]).wait()
    plsc.subcore_barrier()
    base = c * N_ACC + s * ROWS_T
    for k in range(ZCH):
        pltpu.sync_copy(acc.at[pl.ds(s * ROWS_T + k * WIN, WIN)], rows_v.at[0])
        pltpu.sync_copy(rows_v.at[0], s_out.at[pl.ds(base + k * WIN, WIN)])
    tail = ROWS_T - ZCH * WIN
    pltpu.sync_copy(acc.at[pl.ds(s * ROWS_T + ZCH * WIN, tail)],
                    rows_v.at[0, pl.ds(0, tail)])
    pltpu.sync_copy(rows_v.at[0, pl.ds(0, tail)],
                    s_out.at[pl.ds(base + ZCH * WIN, tail)])


def _sc_spmm(table, src32, dst32):
    mesh = plsc.VectorSubcoreMesh(core_axis_name="c", subcore_axis_name="s")
    return pl.kernel(
        _sc_spmm_body,
        out_type=jax.ShapeDtypeStruct((NC * N_ACC, HALF), jnp.float32),
        mesh=mesh,
        compiler_params=pltpu.CompilerParams(use_tc_tiling_on_sc=False),
        scratch_types=[
            pltpu.VMEM_SHARED((N_ACC, HALF), jnp.float32),
            pltpu.VMEM((2, KW, WIN), jnp.int32),
            pltpu.VMEM((2, KW, WIN), jnp.int32),
            pltpu.VMEM((4, WIN, HALF), jnp.float32),
            pltpu.SemaphoreType.DMA,
            pltpu.SemaphoreType.DMA,
            pltpu.SemaphoreType.DMA,
            pltpu.SemaphoreType.DMA,
            pltpu.SemaphoreType.DMA,
            pltpu.SemaphoreType.DMA,
            pltpu.SemaphoreType.DMA,
            pltpu.SemaphoreType.DMA,
        ],
    )(table, src32, dst32)


def _tc_prep_body(p0_ref, p1_ref, x_ref, w_ref, dinv_ref, t_ref):
    deg = 1.0 + p0_ref[...] + p1_ref[...]
    dv = lax.rsqrt(deg)
    t = dv * jnp.dot(x_ref[...], w_ref[...], preferred_element_type=jnp.float32)
    dinv_ref[...] = dv
    t_ref[0] = t[:, :HALF]
    t_ref[1] = t[:, HALF:]


def _tc_prep(p0, p1, x, w0):
    return pl.pallas_call(
        _tc_prep_body,
        grid=(G_TC,),
        in_specs=[
            pl.BlockSpec((R_TC, 1), lambda i: (i, 0)),
            pl.BlockSpec((R_TC, 1), lambda i: (i, 0)),
            pl.BlockSpec((R_TC, DIN), lambda i: (i, 0)),
            pl.BlockSpec((DIN, DH), lambda i: (0, 0)),
        ],
        out_specs=[
            pl.BlockSpec((R_TC, 1), lambda i: (i, 0)),
            pl.BlockSpec((NC, R_TC, HALF), lambda i: (0, i, 0)),
        ],
        out_shape=[
            jax.ShapeDtypeStruct((N, 1), jnp.float32),
            jax.ShapeDtypeStruct((NC, N, HALF), jnp.float32),
        ],
    )(p0, p1, x, w0)


def _tc_layer_body(s_ref, tp_ref, dinv_ref, w_ref, b_ref, t_ref):
    u = jnp.concatenate([s_ref[0] + tp_ref[0], s_ref[1] + tp_ref[1]], axis=1)
    dv = dinv_ref[...]
    h = jnp.maximum(dv * u + b_ref[...], 0.0)
    t = dv * jnp.dot(h, w_ref[...], preferred_element_type=jnp.float32)
    t_ref[0] = t[:, :HALF]
    t_ref[1] = t[:, HALF:]


def _tc_layer(s, tp, dinv, w, b):
    return pl.pallas_call(
        _tc_layer_body,
        grid=(G_TC,),
        in_specs=[
            pl.BlockSpec((NC, R_TC, HALF), lambda i: (0, i, 0)),
            pl.BlockSpec((NC, R_TC, HALF), lambda i: (0, i, 0)),
            pl.BlockSpec((R_TC, 1), lambda i: (i, 0)),
            pl.BlockSpec((DH, DH), lambda i: (0, 0)),
            pl.BlockSpec((1, DH), lambda i: (0, 0)),
        ],
        out_specs=pl.BlockSpec((NC, R_TC, HALF), lambda i: (0, i, 0)),
        out_shape=jax.ShapeDtypeStruct((NC, N, HALF), jnp.float32),
    )(s, tp, dinv, w, b)


def _tc_final_body(s_ref, tp_ref, dinv_ref, b_ref, batch_ref, wout_ref,
                   bout_ref, o_ref, pool_acc):
    i = pl.program_id(0)
    u = jnp.concatenate([s_ref[0] + tp_ref[0], s_ref[1] + tp_ref[1]], axis=1)
    dv = dinv_ref[...]
    h = jnp.maximum(dv * u + b_ref[...], 0.0)
    hh = jnp.concatenate([h, jnp.ones((R_TC, 1), jnp.float32)], axis=1)
    oh = (batch_ref[...] == lax.broadcasted_iota(jnp.int32, (1, B), 1))
    oh = oh.astype(jnp.float32)
    pp = lax.dot_general(oh, hh, (((0,), (0,)), ((), ())),
                         preferred_element_type=jnp.float32)

    @pl.when(i == 0)
    def _():
        pool_acc[...] = pp

    @pl.when(i > 0)
    def _():
        pool_acc[...] += pp

    @pl.when(i == G_TC - 1)
    def _():
        cnt = pool_acc[:, DH:DH + 1]
        scl = 1.0 / (jnp.maximum(cnt, 1.0) * jnp.sqrt(cnt + 1e-6))
        o_ref[...] = jnp.dot(pool_acc[:, :DH] * scl, wout_ref[...],
                             preferred_element_type=jnp.float32) + bout_ref[...]


def _tc_final(s, tp, dinv, b, batch2, wout, bout):
    return pl.pallas_call(
        _tc_final_body,
        grid=(G_TC,),
        in_specs=[
            pl.BlockSpec((NC, R_TC, HALF), lambda i: (0, i, 0)),
            pl.BlockSpec((NC, R_TC, HALF), lambda i: (0, i, 0)),
            pl.BlockSpec((R_TC, 1), lambda i: (i, 0)),
            pl.BlockSpec((1, DH), lambda i: (0, 0)),
            pl.BlockSpec((R_TC, 1), lambda i: (i, 0)),
            pl.BlockSpec((DH, DOUT), lambda i: (0, 0)),
            pl.BlockSpec((1, DOUT), lambda i: (0, 0)),
        ],
        out_specs=pl.BlockSpec((B, DOUT), lambda i: (0, 0)),
        out_shape=jax.ShapeDtypeStruct((B, DOUT), jnp.float32),
        scratch_shapes=[pltpu.VMEM((B, DH + 1), jnp.float32)],
    )(s, tp, dinv, b, batch2, wout, bout)


def kernel(x, edge_index, batch, W0, b0, W1, b1, W2, b2, W3, b3, Wout, bout):
    src = edge_index[0]
    dst = edge_index[1]
    pad_i = jnp.arange(PAD, dtype=jnp.int32)
    src_p = jnp.concatenate([src, pad_i % np.int32(N)])
    dst_p = jnp.concatenate([dst, N + (pad_i % np.int32(16))])

    dst_deg = dst_p.reshape(NT, W_DEG, WIN)
    src_t = src_p.reshape(1, NS, W_SP, WIN)
    src_sp = jnp.concatenate([src_t, src_t + N], axis=0).reshape(NT, W_SP, WIN)
    dst_sp = jnp.broadcast_to(dst_p.reshape(1, NS, W_SP, WIN),
                              (NC, NS, W_SP, WIN)).reshape(NT, W_SP, WIN)

    deg_raw = _sc_deg(dst_deg)
    p0 = deg_raw[:N].reshape(N, 1)
    p1 = deg_raw[N_ACC:N_ACC + N].reshape(N, 1)

    dinv, t = _tc_prep(p0, p1, x, W0)
    for (w, b) in ((W1, b0), (W2, b1), (W3, b2)):
        s = _sc_spmm(t.reshape(NC * N, HALF), src_sp, dst_sp)
        t = _tc_layer(s.reshape(NC, N_ACC, HALF), t, dinv, w, b.reshape(1, DH))
    s = _sc_spmm(t.reshape(NC * N, HALF), src_sp, dst_sp)
    return _tc_final(s.reshape(NC, N_ACC, HALF), t, dinv, b3.reshape(1, DH),
                     batch.reshape(N, 1).astype(jnp.int32), Wout,
                     bout.reshape(1, DOUT))


# speedup vs baseline: 25.2464x; 1.2021x over previous
import jax
import jax.numpy as jnp
import numpy as np
from jax import lax
from jax.experimental import pallas as pl
from jax.experimental.pallas import tpu as pltpu
from jax.experimental.pallas import tpu_sc as plsc

N = 50000
E = 800000
B = 16
DIN = 128
DH = 64
DOUT = 64
HALF = 32

NC = 2
NS = 16
NT = NC * NS

WIN = 128
EP = 802816
PAD = EP - E
W_DEG = EP // (NT * WIN)
W_SP = EP // (NS * WIN)
KW = 8

ROWS_T = 3136
N_ACC = NS * ROWS_T
ZCH = 24

R_TC = 1000
G_TC = N // R_TC


def _fill_zeros_1d(ref, n):
    zero = jnp.zeros((16,), jnp.float32)
    for i in range(n // 16):
        ref[pl.ds(i * 16, 16)] = zero


def _sc_deg_body(dst_hbm, deg_out, acc, idx_v, ones_v, zeros_v):
    c = lax.axis_index("c")
    s = lax.axis_index("s")
    g = c * NS + s
    one = jnp.ones((16,), jnp.float32)
    for i in range(WIN // 16):
        ones_v[pl.ds(i * 16, 16)] = one
    _fill_zeros_1d(zeros_v, ROWS_T)
    pltpu.sync_copy(dst_hbm.at[g], idx_v)
    pltpu.sync_copy(zeros_v, acc.at[pl.ds(s * ROWS_T, ROWS_T)])
    plsc.subcore_barrier()

    def w_body(w, carry):
        pltpu.sync_copy(ones_v, acc.at[idx_v.at[w]], add=True)
        return carry

    lax.fori_loop(0, W_DEG, w_body, 0)
    plsc.subcore_barrier()
    pltpu.sync_copy(acc.at[pl.ds(s * ROWS_T, ROWS_T)], zeros_v)
    pltpu.sync_copy(zeros_v, deg_out.at[pl.ds(c * N_ACC + s * ROWS_T, ROWS_T)])


def _sc_deg(dst32):
    mesh = plsc.VectorSubcoreMesh(core_axis_name="c", subcore_axis_name="s")
    return pl.kernel(
        _sc_deg_body,
        out_type=jax.ShapeDtypeStruct((NC * N_ACC,), jnp.float32),
        mesh=mesh,
        scratch_types=[
            pltpu.VMEM_SHARED((N_ACC,), jnp.float32),
            pltpu.VMEM((W_DEG, WIN), jnp.int32),
            pltpu.VMEM((WIN,), jnp.float32),
            pltpu.VMEM((ROWS_T,), jnp.float32),
        ],
    )(dst32)


def _sc_spmm_body(table_hbm, src_hbm, dst_hbm, s_out, acc, sidx_v, didx_v,
                  rows_v, g0, g1, g2, g3, s0, s1, s2, s3):
    c = lax.axis_index("c")
    s = lax.axis_index("s")
    g = c * NS + s
    gsems = (g0, g1, g2, g3)
    ssems = (s0, s1, s2, s3)
    zero = jnp.zeros((16,), jnp.float32)
    for i in range(WIN):
        rows_v[0, i, pl.ds(0, 16)] = zero
        rows_v[0, i, pl.ds(16, 16)] = zero
    for k in range(ZCH):
        pltpu.sync_copy(rows_v.at[0], acc.at[pl.ds(s * ROWS_T + k * WIN, WIN)])
    pltpu.sync_copy(rows_v.at[0, pl.ds(0, ROWS_T - ZCH * WIN)],
                    acc.at[pl.ds(s * ROWS_T + ZCH * WIN, ROWS_T - ZCH * WIN)])
    plsc.subcore_barrier()

    n_blk = W_SP // KW
    pltpu.sync_copy(src_hbm.at[g, pl.ds(0, KW)], sidx_v.at[0])
    pltpu.sync_copy(dst_hbm.at[g, pl.ds(0, KW)], didx_v.at[0])
    pltpu.async_copy(table_hbm.at[sidx_v.at[0, 0]], rows_v.at[0], gsems[0])
    pltpu.async_copy(table_hbm.at[sidx_v.at[0, 1]], rows_v.at[1], gsems[1])

    def blk_body(blk, carry):
        rb = blk % 2
        for j in range(KW):
            b = j % 4
            bp2 = (j + 2) % 4
            pltpu.make_async_copy(table_hbm.at[sidx_v.at[rb, j]],
                                  rows_v.at[b], gsems[b]).wait()
            if j >= 2:
                pltpu.make_async_copy(rows_v.at[bp2],
                                      acc.at[didx_v.at[rb, j - 2]],
                                      ssems[bp2]).wait()
            else:
                @pl.when(blk > 0)
                def _():
                    pltpu.make_async_copy(rows_v.at[bp2],
                                          acc.at[didx_v.at[rb, j]],
                                          ssems[bp2]).wait()
            if j == 2:
                @pl.when(blk + 1 < n_blk)
                def _():
                    pltpu.sync_copy(src_hbm.at[g, pl.ds((blk + 1) * KW, KW)],
                                    sidx_v.at[1 - rb])
                    pltpu.sync_copy(dst_hbm.at[g, pl.ds((blk + 1) * KW, KW)],
                                    didx_v.at[1 - rb])
            if j < KW - 2:
                pltpu.async_copy(table_hbm.at[sidx_v.at[rb, j + 2]],
                                 rows_v.at[bp2], gsems[bp2])
            else:
                @pl.when(blk + 1 < n_blk)
                def _():
                    pltpu.async_copy(table_hbm.at[sidx_v.at[1 - rb, j - (KW - 2)]],
                                     rows_v.at[bp2], gsems[bp2])
            pltpu.async_copy(rows_v.at[b], acc.at[didx_v.at[rb, j]], ssems[b],
                             add=True)
        return carry

    lax.fori_loop(0, n_blk, blk_body, 0)
    rbl = (n_blk - 1) % 2
    pltpu.make_async_copy(rows_v.at[2], acc.at[didx_v.at[rbl, KW - 2]],
                          ssems[2]).wait()
    pltpu.make_async_copy(rows_v.at[3], acc.at[didx_v.at[rbl, KW - 1]],
                          ssems[3]).wait()
    plsc.subcore_barrier()
    base = c * N_ACC + s * ROWS_T
    for k in range(ZCH):
        pltpu.sync_copy(acc.at[pl.ds(s * ROWS_T + k * WIN, WIN)], rows_v.at[0])
        pltpu.sync_copy(rows_v.at[0], s_out.at[pl.ds(base + k * WIN, WIN)])
    tail = ROWS_T - ZCH * WIN
    pltpu.sync_copy(acc.at[pl.ds(s * ROWS_T + ZCH * WIN, tail)],
                    rows_v.at[0, pl.ds(0, tail)])
    pltpu.sync_copy(rows_v.at[0, pl.ds(0, tail)],
                    s_out.at[pl.ds(base + ZCH * WIN, tail)])


def _sc_spmm(table, src32, dst32):
    mesh = plsc.VectorSubcoreMesh(core_axis_name="c", subcore_axis_name="s")
    return pl.kernel(
        _sc_spmm_body,
        out_type=jax.ShapeDtypeStruct((NC * N_ACC, HALF), jnp.float32),
        mesh=mesh,
        compiler_params=pltpu.CompilerParams(use_tc_tiling_on_sc=False),
        scratch_types=[
            pltpu.VMEM_SHARED((N_ACC, HALF), jnp.float32),
            pltpu.VMEM((2, KW, WIN), jnp.int32),
            pltpu.VMEM((2, KW, WIN), jnp.int32),
            pltpu.VMEM((4, WIN, HALF), jnp.float32),
            pltpu.SemaphoreType.DMA,
            pltpu.SemaphoreType.DMA,
            pltpu.SemaphoreType.DMA,
            pltpu.SemaphoreType.DMA,
            pltpu.SemaphoreType.DMA,
            pltpu.SemaphoreType.DMA,
            pltpu.SemaphoreType.DMA,
            pltpu.SemaphoreType.DMA,
        ],
    )(table, src32, dst32)


NP = N // 4
NP_ACC = N_ACC // 4
R_P = 224


def _tc_prep_body(p0_ref, p1_ref, x_ref, m0_ref, dvp_ref, t_ref):
    dv = lax.rsqrt(1.0 + p0_ref[...] + p1_ref[...])
    xb = x_ref[...]
    dvp_ref[...] = dv
    t_ref[0] = dv * jnp.dot(xb, m0_ref[0], preferred_element_type=jnp.float32)
    t_ref[1] = dv * jnp.dot(xb, m0_ref[1], preferred_element_type=jnp.float32)


def _tc_prep(p0e, p1e, x_pack, m0):
    return pl.pallas_call(
        _tc_prep_body,
        grid=(NP_ACC // R_P,),
        in_specs=[
            pl.BlockSpec((R_P, 128), lambda i: (i, 0)),
            pl.BlockSpec((R_P, 128), lambda i: (i, 0)),
            pl.BlockSpec((R_P, 4 * DIN), lambda i: (i, 0)),
            pl.BlockSpec((NC, 4 * DIN, 128), lambda i: (0, 0, 0)),
        ],
        out_specs=[
            pl.BlockSpec((R_P, 128), lambda i: (i, 0)),
            pl.BlockSpec((NC, R_P, 128), lambda i: (0, i, 0)),
        ],
        out_shape=[
            jax.ShapeDtypeStruct((NP_ACC, 128), jnp.float32),
            jax.ShapeDtypeStruct((NC, NP_ACC, 128), jnp.float32),
        ],
    )(p0e, p1e, x_pack, m0)


def _tc_layer_body(s_ref, tp_ref, dvp_ref, m_ref, b_ref, t_ref):
    dv = dvp_ref[...]
    h0 = jnp.maximum(dv * (s_ref[0] + tp_ref[0]) + b_ref[0], 0.0)
    h1 = jnp.maximum(dv * (s_ref[1] + tp_ref[1]) + b_ref[1], 0.0)
    t_ref[0] = dv * (jnp.dot(h0, m_ref[0, 0], preferred_element_type=jnp.float32)
                     + jnp.dot(h1, m_ref[1, 0], preferred_element_type=jnp.float32))
    t_ref[1] = dv * (jnp.dot(h0, m_ref[0, 1], preferred_element_type=jnp.float32)
                     + jnp.dot(h1, m_ref[1, 1], preferred_element_type=jnp.float32))


def _tc_layer(s_pack, tp, dvp, m, b_pack):
    return pl.pallas_call(
        _tc_layer_body,
        grid=(NP_ACC // R_P,),
        in_specs=[
            pl.BlockSpec((NC, R_P, 128), lambda i: (0, i, 0)),
            pl.BlockSpec((NC, R_P, 128), lambda i: (0, i, 0)),
            pl.BlockSpec((R_P, 128), lambda i: (i, 0)),
            pl.BlockSpec((NC, NC, 128, 128), lambda i: (0, 0, 0, 0)),
            pl.BlockSpec((NC, 1, 128), lambda i: (0, 0, 0)),
        ],
        out_specs=pl.BlockSpec((NC, R_P, 128), lambda i: (0, i, 0)),
        out_shape=jax.ShapeDtypeStruct((NC, NP_ACC, 128), jnp.float32),
    )(s_pack, tp, dvp, m, b_pack)


def _tc_final_body(s_ref, tp_ref, dinv_ref, b_ref, batch_ref, wout_ref,
                   bout_ref, o_ref, pool_acc):
    i = pl.program_id(0)
    u = jnp.concatenate([s_ref[0] + tp_ref[0], s_ref[1] + tp_ref[1]], axis=1)
    dv = dinv_ref[...]
    h = jnp.maximum(dv * u + b_ref[...], 0.0)
    hh = jnp.concatenate([h, jnp.ones((R_TC, 1), jnp.float32)], axis=1)
    oh = (batch_ref[...] == lax.broadcasted_iota(jnp.int32, (1, B), 1))
    oh = oh.astype(jnp.float32)
    pp = lax.dot_general(oh, hh, (((0,), (0,)), ((), ())),
                         preferred_element_type=jnp.float32)

    @pl.when(i == 0)
    def _():
        pool_acc[...] = pp

    @pl.when(i > 0)
    def _():
        pool_acc[...] += pp

    @pl.when(i == G_TC - 1)
    def _():
        cnt = pool_acc[:, DH:DH + 1]
        scl = 1.0 / (jnp.maximum(cnt, 1.0) * jnp.sqrt(cnt + 1e-6))
        o_ref[...] = jnp.dot(pool_acc[:, :DH] * scl, wout_ref[...],
                             preferred_element_type=jnp.float32) + bout_ref[...]


def _tc_final(s, tp, dinv, b, batch2, wout, bout):
    return pl.pallas_call(
        _tc_final_body,
        grid=(G_TC,),
        in_specs=[
            pl.BlockSpec((NC, R_TC, HALF), lambda i: (0, i, 0)),
            pl.BlockSpec((NC, R_TC, HALF), lambda i: (0, i, 0)),
            pl.BlockSpec((R_TC, 1), lambda i: (i, 0)),
            pl.BlockSpec((1, DH), lambda i: (0, 0)),
            pl.BlockSpec((R_TC, 1), lambda i: (i, 0)),
            pl.BlockSpec((DH, DOUT), lambda i: (0, 0)),
            pl.BlockSpec((1, DOUT), lambda i: (0, 0)),
        ],
        out_specs=pl.BlockSpec((B, DOUT), lambda i: (0, 0)),
        out_shape=jax.ShapeDtypeStruct((B, DOUT), jnp.float32),
        scratch_shapes=[pltpu.VMEM((B, DH + 1), jnp.float32)],
    )(s, tp, dinv, b, batch2, wout, bout)


def kernel(x, edge_index, batch, W0, b0, W1, b1, W2, b2, W3, b3, Wout, bout):
    src = edge_index[0]
    dst = edge_index[1]
    pad_i = jnp.arange(PAD, dtype=jnp.int32)
    src_p = jnp.concatenate([src, pad_i % np.int32(N)])
    dst_p = jnp.concatenate([dst, N + (pad_i % np.int32(16))])

    dst_deg = dst_p.reshape(NT, W_DEG, WIN)
    src_t = src_p.reshape(1, NS, W_SP, WIN)
    src_sp = jnp.concatenate([src_t, src_t + N_ACC], axis=0).reshape(NT, W_SP, WIN)
    dst_sp = jnp.broadcast_to(dst_p.reshape(1, NS, W_SP, WIN),
                              (NC, NS, W_SP, WIN)).reshape(NT, W_SP, WIN)

    deg_raw = _sc_deg(dst_deg)

    def _expand(p):
        return jnp.broadcast_to(p.reshape(NP_ACC, 4, 1),
                                (NP_ACC, 4, HALF)).reshape(NP_ACC, 128)

    p0e = _expand(deg_raw[:N_ACC])
    p1e = _expand(deg_raw[N_ACC:])
    x_pack = jnp.concatenate(
        [x.reshape(NP, 4 * DIN),
         jnp.zeros((NP_ACC - NP, 4 * DIN), jnp.float32)])

    i4 = jnp.eye(4, dtype=jnp.float32)
    m0 = jnp.stack([jnp.kron(i4, W0[:, :HALF]), jnp.kron(i4, W0[:, HALF:])])
    dvp, t = _tc_prep(p0e, p1e, x_pack, m0)
    for (w, b) in ((W1, b0), (W2, b1), (W3, b2)):
        mw = jnp.stack([
            jnp.stack([jnp.kron(i4, w[:HALF, :HALF]), jnp.kron(i4, w[:HALF, HALF:])]),
            jnp.stack([jnp.kron(i4, w[HALF:, :HALF]), jnp.kron(i4, w[HALF:, HALF:])]),
        ])
        bp = jnp.tile(b.reshape(NC, 1, HALF), (1, 1, 4))
        s = _sc_spmm(t.reshape(NC * N_ACC, HALF), src_sp, dst_sp)
        t = _tc_layer(s.reshape(NC, NP_ACC, 128), t, dvp, mw, bp)
    s = _sc_spmm(t.reshape(NC * N_ACC, HALF), src_sp, dst_sp)
    dinv = dvp[:NP].reshape(NP, 4, HALF)[:, :, :1].reshape(N, 1)
    return _tc_final(s.reshape(NC, N_ACC, HALF), t.reshape(NC, N_ACC, HALF),
                     dinv, b3.reshape(1, DH),
                     batch.reshape(N, 1).astype(jnp.int32),
                     Wout, bout.reshape(1, DOUT))


# speedup vs baseline: 27.9649x; 1.1077x over previous
import jax
import jax.numpy as jnp
import numpy as np
from jax import lax
from jax.experimental import pallas as pl
from jax.experimental.pallas import tpu as pltpu
from jax.experimental.pallas import tpu_sc as plsc

N = 50000
E = 800000
B = 16
DIN = 128
DH = 64
DOUT = 64
HALF = 32

NC = 2
NS = 16
NT = NC * NS

WIN = 128
EP = 802816
PAD = EP - E
W_DEG = EP // (NT * WIN)
W_SP = EP // (NS * WIN)
KW = 28

ROWS_T = 3136
N_ACC = NS * ROWS_T
ZCH = 24

R_TC = 1000
G_TC = N // R_TC


def _fill_zeros_1d(ref, n):
    zero = jnp.zeros((16,), jnp.float32)
    for i in range(n // 16):
        ref[pl.ds(i * 16, 16)] = zero


def _sc_deg_body(dst_hbm, deg_out, acc, idx_v, ones_v, zeros_v):
    c = lax.axis_index("c")
    s = lax.axis_index("s")
    g = c * NS + s
    one = jnp.ones((16,), jnp.float32)
    for i in range(WIN // 16):
        ones_v[pl.ds(i * 16, 16)] = one
    _fill_zeros_1d(zeros_v, ROWS_T)
    pltpu.sync_copy(dst_hbm.at[g], idx_v)
    pltpu.sync_copy(zeros_v, acc.at[pl.ds(s * ROWS_T, ROWS_T)])
    plsc.subcore_barrier()

    def w_body(w, carry):
        pltpu.sync_copy(ones_v, acc.at[idx_v.at[w]], add=True)
        return carry

    lax.fori_loop(0, W_DEG, w_body, 0)
    plsc.subcore_barrier()
    pltpu.sync_copy(acc.at[pl.ds(s * ROWS_T, ROWS_T)], zeros_v)
    pltpu.sync_copy(zeros_v, deg_out.at[pl.ds(c * N_ACC + s * ROWS_T, ROWS_T)])


def _sc_deg(dst32):
    mesh = plsc.VectorSubcoreMesh(core_axis_name="c", subcore_axis_name="s")
    return pl.kernel(
        _sc_deg_body,
        out_type=jax.ShapeDtypeStruct((NC * N_ACC,), jnp.float32),
        mesh=mesh,
        scratch_types=[
            pltpu.VMEM_SHARED((N_ACC,), jnp.float32),
            pltpu.VMEM((W_DEG, WIN), jnp.int32),
            pltpu.VMEM((WIN,), jnp.float32),
            pltpu.VMEM((ROWS_T,), jnp.float32),
        ],
    )(dst32)


def _sc_spmm_body(table_hbm, src_hbm, dst_hbm, s_out, acc, sidx_v, didx_v,
                  rows_v, g0, g1, g2, g3, s0, s1, s2, s3):
    c = lax.axis_index("c")
    s = lax.axis_index("s")
    g = c * NS + s
    gsems = (g0, g1, g2, g3)
    ssems = (s0, s1, s2, s3)
    zero = jnp.zeros((16,), jnp.float32)
    for i in range(WIN):
        rows_v[0, i, pl.ds(0, 16)] = zero
        rows_v[0, i, pl.ds(16, 16)] = zero
    for k in range(ZCH):
        pltpu.sync_copy(rows_v.at[0], acc.at[pl.ds(s * ROWS_T + k * WIN, WIN)])
    pltpu.sync_copy(rows_v.at[0, pl.ds(0, ROWS_T - ZCH * WIN)],
                    acc.at[pl.ds(s * ROWS_T + ZCH * WIN, ROWS_T - ZCH * WIN)])
    plsc.subcore_barrier()

    n_blk = W_SP // KW
    pltpu.sync_copy(src_hbm.at[g, pl.ds(0, KW)], sidx_v.at[0])
    pltpu.sync_copy(dst_hbm.at[g, pl.ds(0, KW)], didx_v.at[0])
    pltpu.async_copy(table_hbm.at[sidx_v.at[0, 0]], rows_v.at[0], gsems[0])
    pltpu.async_copy(table_hbm.at[sidx_v.at[0, 1]], rows_v.at[1], gsems[1])

    def blk_body(blk, carry):
        rb = blk % 2
        for j in range(KW):
            b = j % 4
            bp2 = (j + 2) % 4
            pltpu.make_async_copy(table_hbm.at[sidx_v.at[rb, j]],
                                  rows_v.at[b], gsems[b]).wait()
            if j >= 2:
                pltpu.make_async_copy(rows_v.at[bp2],
                                      acc.at[didx_v.at[rb, j - 2]],
                                      ssems[bp2]).wait()
            else:
                @pl.when(blk > 0)
                def _():
                    pltpu.make_async_copy(rows_v.at[bp2],
                                          acc.at[didx_v.at[rb, j]],
                                          ssems[bp2]).wait()
            if j == 2:
                @pl.when(blk + 1 < n_blk)
                def _():
                    pltpu.sync_copy(src_hbm.at[g, pl.ds((blk + 1) * KW, KW)],
                                    sidx_v.at[1 - rb])
                    pltpu.sync_copy(dst_hbm.at[g, pl.ds((blk + 1) * KW, KW)],
                                    didx_v.at[1 - rb])
            if j < KW - 2:
                pltpu.async_copy(table_hbm.at[sidx_v.at[rb, j + 2]],
                                 rows_v.at[bp2], gsems[bp2])
            else:
                @pl.when(blk + 1 < n_blk)
                def _():
                    pltpu.async_copy(table_hbm.at[sidx_v.at[1 - rb, j - (KW - 2)]],
                                     rows_v.at[bp2], gsems[bp2])
            pltpu.async_copy(rows_v.at[b], acc.at[didx_v.at[rb, j]], ssems[b],
                             add=True)
        return carry

    lax.fori_loop(0, n_blk, blk_body, 0)
    rbl = (n_blk - 1) % 2
    pltpu.make_async_copy(rows_v.at[2], acc.at[didx_v.at[rbl, KW - 2]],
                          ssems[2]).wait()
    pltpu.make_async_copy(rows_v.at[3], acc.at[didx_v.at[rbl, KW - 1]],
                          ssems[3]).wait()
    plsc.subcore_barrier()
    base = c * N_ACC + s * ROWS_T
    for k in range(ZCH):
        pltpu.sync_copy(acc.at[pl.ds(s * ROWS_T + k * WIN, WIN)], rows_v.at[0])
        pltpu.sync_copy(rows_v.at[0], s_out.at[pl.ds(base + k * WIN, WIN)])
    tail = ROWS_T - ZCH * WIN
    pltpu.sync_copy(acc.at[pl.ds(s * ROWS_T + ZCH * WIN, tail)],
                    rows_v.at[0, pl.ds(0, tail)])
    pltpu.sync_copy(rows_v.at[0, pl.ds(0, tail)],
                    s_out.at[pl.ds(base + ZCH * WIN, tail)])


def _sc_spmm(table, src32, dst32):
    mesh = plsc.VectorSubcoreMesh(core_axis_name="c", subcore_axis_name="s")
    return pl.kernel(
        _sc_spmm_body,
        out_type=jax.ShapeDtypeStruct((NC * N_ACC, HALF), jnp.float32),
        mesh=mesh,
        compiler_params=pltpu.CompilerParams(use_tc_tiling_on_sc=False),
        scratch_types=[
            pltpu.VMEM_SHARED((N_ACC, HALF), jnp.float32),
            pltpu.VMEM((2, KW, WIN), jnp.int32),
            pltpu.VMEM((2, KW, WIN), jnp.int32),
            pltpu.VMEM((4, WIN, HALF), jnp.float32),
            pltpu.SemaphoreType.DMA,
            pltpu.SemaphoreType.DMA,
            pltpu.SemaphoreType.DMA,
            pltpu.SemaphoreType.DMA,
            pltpu.SemaphoreType.DMA,
            pltpu.SemaphoreType.DMA,
            pltpu.SemaphoreType.DMA,
            pltpu.SemaphoreType.DMA,
        ],
    )(table, src32, dst32)


NP = N // 4
NP_ACC = N_ACC // 4
R_P = 224


def _tc_prep_body(p0_ref, p1_ref, x_ref, m0_ref, dvp_ref, t_ref):
    dv = lax.rsqrt(1.0 + p0_ref[...] + p1_ref[...])
    xb = x_ref[...]
    dvp_ref[...] = dv
    t_ref[0] = dv * jnp.dot(xb, m0_ref[0], preferred_element_type=jnp.float32)
    t_ref[1] = dv * jnp.dot(xb, m0_ref[1], preferred_element_type=jnp.float32)


def _tc_prep(p0e, p1e, x_pack, m0):
    return pl.pallas_call(
        _tc_prep_body,
        grid=(NP_ACC // R_P,),
        in_specs=[
            pl.BlockSpec((R_P, 128), lambda i: (i, 0)),
            pl.BlockSpec((R_P, 128), lambda i: (i, 0)),
            pl.BlockSpec((R_P, 4 * DIN), lambda i: (i, 0)),
            pl.BlockSpec((NC, 4 * DIN, 128), lambda i: (0, 0, 0)),
        ],
        out_specs=[
            pl.BlockSpec((R_P, 128), lambda i: (i, 0)),
            pl.BlockSpec((NC, R_P, 128), lambda i: (0, i, 0)),
        ],
        out_shape=[
            jax.ShapeDtypeStruct((NP_ACC, 128), jnp.float32),
            jax.ShapeDtypeStruct((NC, NP_ACC, 128), jnp.float32),
        ],
    )(p0e, p1e, x_pack, m0)


def _tc_layer_body(s_ref, tp_ref, dvp_ref, m_ref, b_ref, t_ref):
    dv = dvp_ref[...]
    h0 = jnp.maximum(dv * (s_ref[0] + tp_ref[0]) + b_ref[0], 0.0)
    h1 = jnp.maximum(dv * (s_ref[1] + tp_ref[1]) + b_ref[1], 0.0)
    t_ref[0] = dv * (jnp.dot(h0, m_ref[0, 0], preferred_element_type=jnp.float32)
                     + jnp.dot(h1, m_ref[1, 0], preferred_element_type=jnp.float32))
    t_ref[1] = dv * (jnp.dot(h0, m_ref[0, 1], preferred_element_type=jnp.float32)
                     + jnp.dot(h1, m_ref[1, 1], preferred_element_type=jnp.float32))


def _tc_layer(s_pack, tp, dvp, m, b_pack):
    return pl.pallas_call(
        _tc_layer_body,
        grid=(NP_ACC // R_P,),
        in_specs=[
            pl.BlockSpec((NC, R_P, 128), lambda i: (0, i, 0)),
            pl.BlockSpec((NC, R_P, 128), lambda i: (0, i, 0)),
            pl.BlockSpec((R_P, 128), lambda i: (i, 0)),
            pl.BlockSpec((NC, NC, 128, 128), lambda i: (0, 0, 0, 0)),
            pl.BlockSpec((NC, 1, 128), lambda i: (0, 0, 0)),
        ],
        out_specs=pl.BlockSpec((NC, R_P, 128), lambda i: (0, i, 0)),
        out_shape=jax.ShapeDtypeStruct((NC, NP_ACC, 128), jnp.float32),
    )(s_pack, tp, dvp, m, b_pack)


def _tc_final_body(s_ref, tp_ref, dinv_ref, b_ref, batch_ref, wout_ref,
                   bout_ref, o_ref, pool_acc):
    i = pl.program_id(0)
    u = jnp.concatenate([s_ref[0] + tp_ref[0], s_ref[1] + tp_ref[1]], axis=1)
    dv = dinv_ref[...]
    h = jnp.maximum(dv * u + b_ref[...], 0.0)
    hh = jnp.concatenate([h, jnp.ones((R_TC, 1), jnp.float32)], axis=1)
    oh = (batch_ref[...] == lax.broadcasted_iota(jnp.int32, (1, B), 1))
    oh = oh.astype(jnp.float32)
    pp = lax.dot_general(oh, hh, (((0,), (0,)), ((), ())),
                         preferred_element_type=jnp.float32)

    @pl.when(i == 0)
    def _():
        pool_acc[...] = pp

    @pl.when(i > 0)
    def _():
        pool_acc[...] += pp

    @pl.when(i == G_TC - 1)
    def _():
        cnt = pool_acc[:, DH:DH + 1]
        scl = 1.0 / (jnp.maximum(cnt, 1.0) * jnp.sqrt(cnt + 1e-6))
        o_ref[...] = jnp.dot(pool_acc[:, :DH] * scl, wout_ref[...],
                             preferred_element_type=jnp.float32) + bout_ref[...]


def _tc_final(s, tp, dinv, b, batch2, wout, bout):
    return pl.pallas_call(
        _tc_final_body,
        grid=(G_TC,),
        in_specs=[
            pl.BlockSpec((NC, R_TC, HALF), lambda i: (0, i, 0)),
            pl.BlockSpec((NC, R_TC, HALF), lambda i: (0, i, 0)),
            pl.BlockSpec((R_TC, 1), lambda i: (i, 0)),
            pl.BlockSpec((1, DH), lambda i: (0, 0)),
            pl.BlockSpec((R_TC, 1), lambda i: (i, 0)),
            pl.BlockSpec((DH, DOUT), lambda i: (0, 0)),
            pl.BlockSpec((1, DOUT), lambda i: (0, 0)),
        ],
        out_specs=pl.BlockSpec((B, DOUT), lambda i: (0, 0)),
        out_shape=jax.ShapeDtypeStruct((B, DOUT), jnp.float32),
        scratch_shapes=[pltpu.VMEM((B, DH + 1), jnp.float32)],
    )(s, tp, dinv, b, batch2, wout, bout)


def kernel(x, edge_index, batch, W0, b0, W1, b1, W2, b2, W3, b3, Wout, bout):
    src = edge_index[0]
    dst = edge_index[1]
    pad_i = jnp.arange(PAD, dtype=jnp.int32)
    src_p = jnp.concatenate([src, pad_i % np.int32(N)])
    dst_p = jnp.concatenate([dst, N + (pad_i % np.int32(16))])

    dst_deg = dst_p.reshape(NT, W_DEG, WIN)
    src_t = src_p.reshape(1, NS, W_SP, WIN)
    src_sp = jnp.concatenate([src_t, src_t + N_ACC], axis=0).reshape(NT, W_SP, WIN)
    dst_sp = jnp.broadcast_to(dst_p.reshape(1, NS, W_SP, WIN),
                              (NC, NS, W_SP, WIN)).reshape(NT, W_SP, WIN)

    deg_raw = _sc_deg(dst_deg)

    def _expand(p):
        return jnp.broadcast_to(p.reshape(NP_ACC, 4, 1),
                                (NP_ACC, 4, HALF)).reshape(NP_ACC, 128)

    p0e = _expand(deg_raw[:N_ACC])
    p1e = _expand(deg_raw[N_ACC:])
    x_pack = jnp.concatenate(
        [x.reshape(NP, 4 * DIN),
         jnp.zeros((NP_ACC - NP, 4 * DIN), jnp.float32)])

    i4 = jnp.eye(4, dtype=jnp.float32)
    m0 = jnp.stack([jnp.kron(i4, W0[:, :HALF]), jnp.kron(i4, W0[:, HALF:])])
    dvp, t = _tc_prep(p0e, p1e, x_pack, m0)
    for (w, b) in ((W1, b0), (W2, b1), (W3, b2)):
        mw = jnp.stack([
            jnp.stack([jnp.kron(i4, w[:HALF, :HALF]), jnp.kron(i4, w[:HALF, HALF:])]),
            jnp.stack([jnp.kron(i4, w[HALF:, :HALF]), jnp.kron(i4, w[HALF:, HALF:])]),
        ])
        bp = jnp.tile(b.reshape(NC, 1, HALF), (1, 1, 4))
        s = _sc_spmm(t.reshape(NC * N_ACC, HALF), src_sp, dst_sp)
        t = _tc_layer(s.reshape(NC, NP_ACC, 128), t, dvp, mw, bp)
    s = _sc_spmm(t.reshape(NC * N_ACC, HALF), src_sp, dst_sp)
    dinv = dvp[:NP].reshape(NP, 4, HALF)[:, :, :1].reshape(N, 1)
    return _tc_final(s.reshape(NC, N_ACC, HALF), t.reshape(NC, N_ACC, HALF),
                     dinv, b3.reshape(1, DH),
                     batch.reshape(N, 1).astype(jnp.int32),
                     Wout, bout.reshape(1, DOUT))


# speedup vs baseline: 29.3390x; 1.0491x over previous
import jax
import jax.numpy as jnp
import numpy as np
from jax import lax
from jax.experimental import pallas as pl
from jax.experimental.pallas import tpu as pltpu
from jax.experimental.pallas import tpu_sc as plsc

N = 50000
E = 800000
B = 16
DIN = 128
DH = 64
DOUT = 64
HALF = 32

NC = 2
NS = 16
NT = NC * NS

WIN = 128
EP = 802816
PAD = EP - E
W_DEG = EP // (NT * WIN)
W_SP = EP // (NS * WIN)
KW = 28

ROWS_T = 3136
N_ACC = NS * ROWS_T
ZCH = 24

R_TC = 1000
G_TC = N // R_TC


def _fill_zeros_1d(ref, n):
    zero = jnp.zeros((16,), jnp.float32)
    for i in range(n // 16):
        ref[pl.ds(i * 16, 16)] = zero


def _sc_deg_body(dst_hbm, deg_out, acc, idx_v, ones_v, zeros_v):
    c = lax.axis_index("c")
    s = lax.axis_index("s")
    g = c * NS + s
    one = jnp.ones((16,), jnp.float32)
    for i in range(WIN // 16):
        ones_v[pl.ds(i * 16, 16)] = one
    _fill_zeros_1d(zeros_v, ROWS_T)
    pltpu.sync_copy(dst_hbm.at[g], idx_v)
    pltpu.sync_copy(zeros_v, acc.at[pl.ds(s * ROWS_T, ROWS_T)])
    plsc.subcore_barrier()

    def w_body(w, carry):
        pltpu.sync_copy(ones_v, acc.at[idx_v.at[w]], add=True)
        return carry

    lax.fori_loop(0, W_DEG, w_body, 0)
    plsc.subcore_barrier()
    pltpu.sync_copy(acc.at[pl.ds(s * ROWS_T, ROWS_T)], zeros_v)
    pltpu.sync_copy(zeros_v, deg_out.at[pl.ds(c * N_ACC + s * ROWS_T, ROWS_T)])


def _sc_deg(dst32):
    mesh = plsc.VectorSubcoreMesh(core_axis_name="c", subcore_axis_name="s")
    return pl.kernel(
        _sc_deg_body,
        out_type=jax.ShapeDtypeStruct((NC * N_ACC,), jnp.float32),
        mesh=mesh,
        scratch_types=[
            pltpu.VMEM_SHARED((N_ACC,), jnp.float32),
            pltpu.VMEM((W_DEG, WIN), jnp.int32),
            pltpu.VMEM((WIN,), jnp.float32),
            pltpu.VMEM((ROWS_T,), jnp.float32),
        ],
    )(dst32)


def _sc_spmm_body(table_hbm, src_hbm, dst_hbm, s_out, acc, sidx_v, didx_v,
                  rows_v, g0, g1, g2, g3, s0, s1, s2, s3):
    c = lax.axis_index("c")
    s = lax.axis_index("s")
    g = c * NS + s
    gsems = (g0, g1, g2, g3)
    ssems = (s0, s1, s2, s3)
    zero = jnp.zeros((16,), jnp.float32)
    for i in range(WIN):
        rows_v[0, i, pl.ds(0, 16)] = zero
        rows_v[0, i, pl.ds(16, 16)] = zero
    for k in range(ZCH):
        pltpu.sync_copy(rows_v.at[0], acc.at[pl.ds(s * ROWS_T + k * WIN, WIN)])
    pltpu.sync_copy(rows_v.at[0, pl.ds(0, ROWS_T - ZCH * WIN)],
                    acc.at[pl.ds(s * ROWS_T + ZCH * WIN, ROWS_T - ZCH * WIN)])
    plsc.subcore_barrier()

    n_blk = W_SP // KW
    pltpu.sync_copy(src_hbm.at[g, pl.ds(0, KW)], sidx_v.at[0])
    pltpu.sync_copy(dst_hbm.at[g, pl.ds(0, KW)], didx_v.at[0])
    pltpu.async_copy(table_hbm.at[sidx_v.at[0, 0]], rows_v.at[0], gsems[0])
    pltpu.async_copy(table_hbm.at[sidx_v.at[0, 1]], rows_v.at[1], gsems[1])

    def blk_body(blk, carry):
        rb = blk % 2
        for j in range(KW):
            b = j % 4
            bp2 = (j + 2) % 4
            pltpu.make_async_copy(table_hbm.at[sidx_v.at[rb, j]],
                                  rows_v.at[b], gsems[b]).wait()
            if j >= 2:
                pltpu.make_async_copy(rows_v.at[bp2],
                                      acc.at[didx_v.at[rb, j - 2]],
                                      ssems[bp2]).wait()
            else:
                @pl.when(blk > 0)
                def _():
                    pltpu.make_async_copy(rows_v.at[bp2],
                                          acc.at[didx_v.at[rb, j]],
                                          ssems[bp2]).wait()
            if j == 2:
                @pl.when(blk + 1 < n_blk)
                def _():
                    pltpu.sync_copy(src_hbm.at[g, pl.ds((blk + 1) * KW, KW)],
                                    sidx_v.at[1 - rb])
                    pltpu.sync_copy(dst_hbm.at[g, pl.ds((blk + 1) * KW, KW)],
                                    didx_v.at[1 - rb])
            if j < KW - 2:
                pltpu.async_copy(table_hbm.at[sidx_v.at[rb, j + 2]],
                                 rows_v.at[bp2], gsems[bp2])
            else:
                @pl.when(blk + 1 < n_blk)
                def _():
                    pltpu.async_copy(table_hbm.at[sidx_v.at[1 - rb, j - (KW - 2)]],
                                     rows_v.at[bp2], gsems[bp2])
            pltpu.async_copy(rows_v.at[b], acc.at[didx_v.at[rb, j]], ssems[b],
                             add=True)
        return carry

    lax.fori_loop(0, n_blk, blk_body, 0)
    rbl = (n_blk - 1) % 2
    pltpu.make_async_copy(rows_v.at[2], acc.at[didx_v.at[rbl, KW - 2]],
                          ssems[2]).wait()
    pltpu.make_async_copy(rows_v.at[3], acc.at[didx_v.at[rbl, KW - 1]],
                          ssems[3]).wait()
    plsc.subcore_barrier()
    base = c * N_ACC + s * ROWS_T
    for k in range(ZCH):
        pltpu.sync_copy(acc.at[pl.ds(s * ROWS_T + k * WIN, WIN)], rows_v.at[0])
        pltpu.sync_copy(rows_v.at[0], s_out.at[pl.ds(base + k * WIN, WIN)])
    tail = ROWS_T - ZCH * WIN
    pltpu.sync_copy(acc.at[pl.ds(s * ROWS_T + ZCH * WIN, tail)],
                    rows_v.at[0, pl.ds(0, tail)])
    pltpu.sync_copy(rows_v.at[0, pl.ds(0, tail)],
                    s_out.at[pl.ds(base + ZCH * WIN, tail)])


def _sc_spmm(table, src32, dst32):
    mesh = plsc.VectorSubcoreMesh(core_axis_name="c", subcore_axis_name="s")
    return pl.kernel(
        _sc_spmm_body,
        out_type=jax.ShapeDtypeStruct((NC * N_ACC, HALF), jnp.float32),
        mesh=mesh,
        compiler_params=pltpu.CompilerParams(use_tc_tiling_on_sc=False),
        scratch_types=[
            pltpu.VMEM_SHARED((N_ACC, HALF), jnp.float32),
            pltpu.VMEM((2, KW, WIN), jnp.int32),
            pltpu.VMEM((2, KW, WIN), jnp.int32),
            pltpu.VMEM((4, WIN, HALF), jnp.float32),
            pltpu.SemaphoreType.DMA,
            pltpu.SemaphoreType.DMA,
            pltpu.SemaphoreType.DMA,
            pltpu.SemaphoreType.DMA,
            pltpu.SemaphoreType.DMA,
            pltpu.SemaphoreType.DMA,
            pltpu.SemaphoreType.DMA,
            pltpu.SemaphoreType.DMA,
        ],
    )(table, src32, dst32)


NP = N // 4
NP_ACC = N_ACC // 4
R_P = 224


def _tc_prep_body(p0_ref, p1_ref, x_ref, m0_ref, dvp_ref, t_ref):
    dv = lax.rsqrt(1.0 + p0_ref[...] + p1_ref[...])
    xb = x_ref[...]
    dvp_ref[...] = dv
    t_ref[0] = dv * jnp.dot(xb, m0_ref[0], preferred_element_type=jnp.float32)
    t_ref[1] = dv * jnp.dot(xb, m0_ref[1], preferred_element_type=jnp.float32)


def _tc_prep(p0e, p1e, x_pack, m0):
    return pl.pallas_call(
        _tc_prep_body,
        grid=(NP_ACC // R_P,),
        in_specs=[
            pl.BlockSpec((R_P, 128), lambda i: (i, 0)),
            pl.BlockSpec((R_P, 128), lambda i: (i, 0)),
            pl.BlockSpec((R_P, 4 * DIN), lambda i: (i, 0)),
            pl.BlockSpec((NC, 4 * DIN, 128), lambda i: (0, 0, 0)),
        ],
        out_specs=[
            pl.BlockSpec((R_P, 128), lambda i: (i, 0)),
            pl.BlockSpec((NC, R_P, 128), lambda i: (0, i, 0)),
        ],
        out_shape=[
            jax.ShapeDtypeStruct((NP_ACC, 128), jnp.float32),
            jax.ShapeDtypeStruct((NC, NP_ACC, 128), jnp.float32),
        ],
    )(p0e, p1e, x_pack, m0)


def _tc_layer_body(s_ref, tp_ref, dvp_ref, m_ref, b_ref, t_ref):
    dv = dvp_ref[...]
    h0 = jnp.maximum(dv * (s_ref[0] + tp_ref[0]) + b_ref[0], 0.0)
    h1 = jnp.maximum(dv * (s_ref[1] + tp_ref[1]) + b_ref[1], 0.0)
    t_ref[0] = dv * (jnp.dot(h0, m_ref[0, 0], preferred_element_type=jnp.float32)
                     + jnp.dot(h1, m_ref[1, 0], preferred_element_type=jnp.float32))
    t_ref[1] = dv * (jnp.dot(h0, m_ref[0, 1], preferred_element_type=jnp.float32)
                     + jnp.dot(h1, m_ref[1, 1], preferred_element_type=jnp.float32))


def _tc_layer(s_pack, tp, dvp, m, b_pack):
    return pl.pallas_call(
        _tc_layer_body,
        grid=(NP_ACC // R_P,),
        in_specs=[
            pl.BlockSpec((NC, R_P, 128), lambda i: (0, i, 0)),
            pl.BlockSpec((NC, R_P, 128), lambda i: (0, i, 0)),
            pl.BlockSpec((R_P, 128), lambda i: (i, 0)),
            pl.BlockSpec((NC, NC, 128, 128), lambda i: (0, 0, 0, 0)),
            pl.BlockSpec((NC, 1, 128), lambda i: (0, 0, 0)),
        ],
        out_specs=pl.BlockSpec((NC, R_P, 128), lambda i: (0, i, 0)),
        out_shape=jax.ShapeDtypeStruct((NC, NP_ACC, 128), jnp.float32),
    )(s_pack, tp, dvp, m, b_pack)


def _tc_final_body(s_ref, tp_ref, dvp_ref, b_ref, batch_ref, rowsel_ref,
                   z_ref, wout_ref, bout_ref, o_ref, pool_acc, cnt_acc):
    i = pl.program_id(0)
    n_i = pl.num_programs(0)
    dv = dvp_ref[...]
    oh = (jnp.tile(batch_ref[...], (1, B))
          == lax.broadcasted_iota(jnp.int32, (1, 4 * B), 1) // 4)
    oh = oh.astype(jnp.float32)
    cc = jnp.sum(oh, axis=0, keepdims=True)
    pps = []
    for c in range(NC):
        h = jnp.maximum(dv * (s_ref[c] + tp_ref[c]) + b_ref[c], 0.0)
        pps.append(lax.dot_general(oh, h, (((0,), (0,)), ((), ())),
                                   preferred_element_type=jnp.float32))

    @pl.when(i == 0)
    def _():
        pool_acc[0] = pps[0]
        pool_acc[1] = pps[1]
        cnt_acc[...] = cc

    @pl.when(i > 0)
    def _():
        pool_acc[0] += pps[0]
        pool_acc[1] += pps[1]
        cnt_acc[...] += cc

    @pl.when(i == n_i - 1)
    def _():
        halves = []
        for c in range(NC):
            pc = jnp.zeros((B, HALF), jnp.float32)
            for k in range(4):
                pick = jnp.dot(rowsel_ref[k], pool_acc[c],
                               preferred_element_type=jnp.float32)
                pc = pc + pick[:, k * HALF:(k + 1) * HALF]
            halves.append(pc)
        pooled = jnp.concatenate(halves, axis=1)
        cnt = lax.dot_general(z_ref[...], cnt_acc[...], (((1,), (1,)), ((), ())),
                              preferred_element_type=jnp.float32)
        scl = 1.0 / (jnp.maximum(cnt, 1.0) * jnp.sqrt(cnt + 1e-6))
        o_ref[...] = jnp.dot(pooled * scl, wout_ref[...],
                             preferred_element_type=jnp.float32) + bout_ref[...]


def _tc_final(s_pack, tp, dvp, b_pack, batch_pack, rowsel, z, wout, bout):
    return pl.pallas_call(
        _tc_final_body,
        grid=(NP_ACC // R_P,),
        in_specs=[
            pl.BlockSpec((NC, R_P, 128), lambda i: (0, i, 0)),
            pl.BlockSpec((NC, R_P, 128), lambda i: (0, i, 0)),
            pl.BlockSpec((R_P, 128), lambda i: (i, 0)),
            pl.BlockSpec((NC, 1, 128), lambda i: (0, 0, 0)),
            pl.BlockSpec((R_P, 4), lambda i: (i, 0)),
            pl.BlockSpec((4, B, 4 * B), lambda i: (0, 0, 0)),
            pl.BlockSpec((B, 4 * B), lambda i: (0, 0)),
            pl.BlockSpec((DH, DOUT), lambda i: (0, 0)),
            pl.BlockSpec((1, DOUT), lambda i: (0, 0)),
        ],
        out_specs=pl.BlockSpec((B, DOUT), lambda i: (0, 0)),
        out_shape=jax.ShapeDtypeStruct((B, DOUT), jnp.float32),
        scratch_shapes=[pltpu.VMEM((NC, 4 * B, 128), jnp.float32),
                        pltpu.VMEM((1, 4 * B), jnp.float32)],
    )(s_pack, tp, dvp, b_pack, batch_pack, rowsel, z, wout, bout)


def kernel(x, edge_index, batch, W0, b0, W1, b1, W2, b2, W3, b3, Wout, bout):
    src = edge_index[0]
    dst = edge_index[1]
    pad_i = jnp.arange(PAD, dtype=jnp.int32)
    src_p = jnp.concatenate([src, pad_i % np.int32(N)])
    dst_p = jnp.concatenate([dst, N + (pad_i % np.int32(16))])

    dst_deg = dst_p.reshape(NT, W_DEG, WIN)
    src_t = src_p.reshape(1, NS, W_SP, WIN)
    src_sp = jnp.concatenate([src_t, src_t + N_ACC], axis=0).reshape(NT, W_SP, WIN)
    dst_sp = jnp.broadcast_to(dst_p.reshape(1, NS, W_SP, WIN),
                              (NC, NS, W_SP, WIN)).reshape(NT, W_SP, WIN)

    deg_raw = _sc_deg(dst_deg)

    def _expand(p):
        return jnp.broadcast_to(p.reshape(NP_ACC, 4, 1),
                                (NP_ACC, 4, HALF)).reshape(NP_ACC, 128)

    p0e = _expand(deg_raw[:N_ACC])
    p1e = _expand(deg_raw[N_ACC:])
    x_pack = jnp.concatenate(
        [x.reshape(NP, 4 * DIN),
         jnp.zeros((NP_ACC - NP, 4 * DIN), jnp.float32)])

    i4 = jnp.eye(4, dtype=jnp.float32)
    m0 = jnp.stack([jnp.kron(i4, W0[:, :HALF]), jnp.kron(i4, W0[:, HALF:])])
    dvp, t = _tc_prep(p0e, p1e, x_pack, m0)
    for (w, b) in ((W1, b0), (W2, b1), (W3, b2)):
        mw = jnp.stack([
            jnp.stack([jnp.kron(i4, w[:HALF, :HALF]), jnp.kron(i4, w[:HALF, HALF:])]),
            jnp.stack([jnp.kron(i4, w[HALF:, :HALF]), jnp.kron(i4, w[HALF:, HALF:])]),
        ])
        bp = jnp.tile(b.reshape(NC, 1, HALF), (1, 1, 4))
        s = _sc_spmm(t.reshape(NC * N_ACC, HALF), src_sp, dst_sp)
        t = _tc_layer(s.reshape(NC, NP_ACC, 128), t, dvp, mw, bp)
    s = _sc_spmm(t.reshape(NC * N_ACC, HALF), src_sp, dst_sp)
    batch_pack = jnp.concatenate(
        [batch.astype(jnp.int32).reshape(NP, 4),
         jnp.full((NP_ACC - NP, 4), -1, jnp.int32)])
    eye_b = jnp.eye(B, dtype=jnp.float32)
    rowsel = jnp.stack([jnp.kron(eye_b, jnp.eye(4, dtype=jnp.float32)[k:k + 1])
                        for k in range(4)])
    z = jnp.kron(eye_b, jnp.ones((1, 4), jnp.float32))
    return _tc_final(s.reshape(NC, NP_ACC, 128), t, dvp,
                     jnp.tile(b3.reshape(NC, 1, HALF), (1, 1, 4)), batch_pack,
                     rowsel, z, Wout, bout.reshape(1, DOUT))


# speedup vs baseline: 30.2659x; 1.0316x over previous
import jax
import jax.numpy as jnp
import numpy as np
from jax import lax
from jax.experimental import pallas as pl
from jax.experimental.pallas import tpu as pltpu
from jax.experimental.pallas import tpu_sc as plsc

N = 50000
E = 800000
B = 16
DIN = 128
DH = 64
DOUT = 64
HALF = 32

NC = 2
NS = 16
NT = NC * NS

WIN = 128
EP = 802816
PAD = EP - E
W_DEG = EP // (NT * WIN)
W_SP = EP // (NS * WIN)
KW = 28

ROWS_T = 3136
N_ACC = NS * ROWS_T
ZCH = 24

R_TC = 1000
G_TC = N // R_TC


def _fill_zeros_1d(ref, n):
    zero = jnp.zeros((16,), jnp.float32)
    for i in range(n // 16):
        ref[pl.ds(i * 16, 16)] = zero


def _sc_deg_body(dst_hbm, deg_out, acc, idx_v, ones_v, zeros_v):
    c = lax.axis_index("c")
    s = lax.axis_index("s")
    g = c * NS + s
    one = jnp.ones((16,), jnp.float32)
    for i in range(WIN // 16):
        ones_v[pl.ds(i * 16, 16)] = one
    _fill_zeros_1d(zeros_v, ROWS_T)
    pltpu.sync_copy(dst_hbm.at[g], idx_v)
    pltpu.sync_copy(zeros_v, acc.at[pl.ds(s * ROWS_T, ROWS_T)])
    plsc.subcore_barrier()

    def w_body(w, carry):
        pltpu.sync_copy(ones_v, acc.at[idx_v.at[w]], add=True)
        return carry

    lax.fori_loop(0, W_DEG, w_body, 0)
    plsc.subcore_barrier()
    pltpu.sync_copy(acc.at[pl.ds(s * ROWS_T, ROWS_T)], zeros_v)
    pltpu.sync_copy(zeros_v, deg_out.at[pl.ds(c * N_ACC + s * ROWS_T, ROWS_T)])


def _sc_deg(dst32):
    mesh = plsc.VectorSubcoreMesh(core_axis_name="c", subcore_axis_name="s")
    return pl.kernel(
        _sc_deg_body,
        out_type=jax.ShapeDtypeStruct((NC * N_ACC,), jnp.float32),
        mesh=mesh,
        scratch_types=[
            pltpu.VMEM_SHARED((N_ACC,), jnp.float32),
            pltpu.VMEM((W_DEG, WIN), jnp.int32),
            pltpu.VMEM((WIN,), jnp.float32),
            pltpu.VMEM((ROWS_T,), jnp.float32),
        ],
    )(dst32)


def _sc_spmm_body(table_hbm, src_hbm, dst_hbm, s_out, acc, sidx_v, didx_v,
                  rows_v, g0, g1, g2, g3, s0, s1, s2, s3):
    c = lax.axis_index("c")
    s = lax.axis_index("s")
    g = c * NS + s
    gsems = (g0, g1, g2, g3)
    ssems = (s0, s1, s2, s3)
    zero = jnp.zeros((16,), jnp.float32)
    for i in range(WIN):
        rows_v[0, i, pl.ds(0, 16)] = zero
        rows_v[0, i, pl.ds(16, 16)] = zero
    for k in range(ZCH):
        pltpu.sync_copy(rows_v.at[0], acc.at[pl.ds(s * ROWS_T + k * WIN, WIN)])
    pltpu.sync_copy(rows_v.at[0, pl.ds(0, ROWS_T - ZCH * WIN)],
                    acc.at[pl.ds(s * ROWS_T + ZCH * WIN, ROWS_T - ZCH * WIN)])
    plsc.subcore_barrier()

    n_blk = W_SP // KW
    pltpu.sync_copy(src_hbm.at[g, pl.ds(0, KW)], sidx_v.at[0])
    pltpu.sync_copy(dst_hbm.at[s, pl.ds(0, KW)], didx_v.at[0])
    pltpu.async_copy(table_hbm.at[sidx_v.at[0, 0]], rows_v.at[0], gsems[0])
    pltpu.async_copy(table_hbm.at[sidx_v.at[0, 1]], rows_v.at[1], gsems[1])

    def blk_body(blk, carry):
        rb = blk % 2
        for j in range(KW):
            b = j % 4
            bp2 = (j + 2) % 4
            pltpu.make_async_copy(table_hbm.at[sidx_v.at[rb, j]],
                                  rows_v.at[b], gsems[b]).wait()
            if j >= 2:
                pltpu.make_async_copy(rows_v.at[bp2],
                                      acc.at[didx_v.at[rb, j - 2]],
                                      ssems[bp2]).wait()
            else:
                @pl.when(blk > 0)
                def _():
                    pltpu.make_async_copy(rows_v.at[bp2],
                                          acc.at[didx_v.at[rb, j]],
                                          ssems[bp2]).wait()
            if j == 2:
                @pl.when(blk + 1 < n_blk)
                def _():
                    pltpu.sync_copy(src_hbm.at[g, pl.ds((blk + 1) * KW, KW)],
                                    sidx_v.at[1 - rb])
                    pltpu.sync_copy(dst_hbm.at[s, pl.ds((blk + 1) * KW, KW)],
                                    didx_v.at[1 - rb])
            if j < KW - 2:
                pltpu.async_copy(table_hbm.at[sidx_v.at[rb, j + 2]],
                                 rows_v.at[bp2], gsems[bp2])
            else:
                @pl.when(blk + 1 < n_blk)
                def _():
                    pltpu.async_copy(table_hbm.at[sidx_v.at[1 - rb, j - (KW - 2)]],
                                     rows_v.at[bp2], gsems[bp2])
            pltpu.async_copy(rows_v.at[b], acc.at[didx_v.at[rb, j]], ssems[b],
                             add=True)
        return carry

    lax.fori_loop(0, n_blk, blk_body, 0)
    rbl = (n_blk - 1) % 2
    pltpu.make_async_copy(rows_v.at[2], acc.at[didx_v.at[rbl, KW - 2]],
                          ssems[2]).wait()
    pltpu.make_async_copy(rows_v.at[3], acc.at[didx_v.at[rbl, KW - 1]],
                          ssems[3]).wait()
    plsc.subcore_barrier()
    base = c * N_ACC + s * ROWS_T
    for k in range(ZCH):
        pltpu.sync_copy(acc.at[pl.ds(s * ROWS_T + k * WIN, WIN)], rows_v.at[0])
        pltpu.sync_copy(rows_v.at[0], s_out.at[pl.ds(base + k * WIN, WIN)])
    tail = ROWS_T - ZCH * WIN
    pltpu.sync_copy(acc.at[pl.ds(s * ROWS_T + ZCH * WIN, tail)],
                    rows_v.at[0, pl.ds(0, tail)])
    pltpu.sync_copy(rows_v.at[0, pl.ds(0, tail)],
                    s_out.at[pl.ds(base + ZCH * WIN, tail)])


def _sc_spmm(table, src32, dst32):
    mesh = plsc.VectorSubcoreMesh(core_axis_name="c", subcore_axis_name="s")
    return pl.kernel(
        _sc_spmm_body,
        out_type=jax.ShapeDtypeStruct((NC * N_ACC, HALF), jnp.float32),
        mesh=mesh,
        compiler_params=pltpu.CompilerParams(use_tc_tiling_on_sc=False),
        scratch_types=[
            pltpu.VMEM_SHARED((N_ACC, HALF), jnp.float32),
            pltpu.VMEM((2, KW, WIN), jnp.int32),
            pltpu.VMEM((2, KW, WIN), jnp.int32),
            pltpu.VMEM((4, WIN, HALF), jnp.float32),
            pltpu.SemaphoreType.DMA,
            pltpu.SemaphoreType.DMA,
            pltpu.SemaphoreType.DMA,
            pltpu.SemaphoreType.DMA,
            pltpu.SemaphoreType.DMA,
            pltpu.SemaphoreType.DMA,
            pltpu.SemaphoreType.DMA,
            pltpu.SemaphoreType.DMA,
        ],
    )(table, src32, dst32)


NP = N // 4
NP_ACC = N_ACC // 4
R_P = 224


def _tc_prep_body(pe_ref, x_ref, m0_ref, dvp_ref, t_ref):
    dv = lax.rsqrt(1.0 + pe_ref[...])
    tt = jnp.dot(x_ref[...], m0_ref[...], preferred_element_type=jnp.float32)
    dvp_ref[...] = dv
    t_ref[0] = dv * tt[:, :128]
    t_ref[1] = dv * tt[:, 128:]


def _tc_prep(pe, x_pack, m0):
    last = NP // R_P - 1
    return pl.pallas_call(
        _tc_prep_body,
        grid=(NP_ACC // R_P,),
        in_specs=[
            pl.BlockSpec((R_P, 128), lambda i: (i, 0)),
            pl.BlockSpec((R_P, 4 * DIN), lambda i: (jnp.minimum(i, last), 0)),
            pl.BlockSpec((4 * DIN, 2 * 128), lambda i: (0, 0)),
        ],
        out_specs=[
            pl.BlockSpec((R_P, 128), lambda i: (i, 0)),
            pl.BlockSpec((NC, R_P, 128), lambda i: (0, i, 0)),
        ],
        out_shape=[
            jax.ShapeDtypeStruct((NP_ACC, 128), jnp.float32),
            jax.ShapeDtypeStruct((NC, NP_ACC, 128), jnp.float32),
        ],
    )(pe, x_pack, m0)


def _tc_layer_body(s_ref, tp_ref, dvp_ref, m_ref, b_ref, t_ref):
    dv = dvp_ref[...]
    h = jnp.concatenate(
        [jnp.maximum(dv * (s_ref[0] + tp_ref[0]) + b_ref[0], 0.0),
         jnp.maximum(dv * (s_ref[1] + tp_ref[1]) + b_ref[1], 0.0)], axis=1)
    tt = jnp.dot(h, m_ref[...], preferred_element_type=jnp.float32)
    t_ref[0] = dv * tt[:, :128]
    t_ref[1] = dv * tt[:, 128:]


def _tc_layer(s_pack, tp, dvp, m, b_pack):
    return pl.pallas_call(
        _tc_layer_body,
        grid=(NP_ACC // R_P,),
        in_specs=[
            pl.BlockSpec((NC, R_P, 128), lambda i: (0, i, 0)),
            pl.BlockSpec((NC, R_P, 128), lambda i: (0, i, 0)),
            pl.BlockSpec((R_P, 128), lambda i: (i, 0)),
            pl.BlockSpec((2 * 128, 2 * 128), lambda i: (0, 0)),
            pl.BlockSpec((NC, 1, 128), lambda i: (0, 0, 0)),
        ],
        out_specs=pl.BlockSpec((NC, R_P, 128), lambda i: (0, i, 0)),
        out_shape=jax.ShapeDtypeStruct((NC, NP_ACC, 128), jnp.float32),
    )(s_pack, tp, dvp, m, b_pack)


def _tc_final_body(s_ref, tp_ref, dvp_ref, b_ref, batch_ref, rowsel_ref,
                   z_ref, wout_ref, bout_ref, o_ref, pool_acc, cnt_acc):
    i = pl.program_id(0)
    n_i = pl.num_programs(0)
    dv = dvp_ref[...]
    oh = (jnp.tile(batch_ref[...], (1, B))
          == lax.broadcasted_iota(jnp.int32, (1, 4 * B), 1) // 4)
    oh = oh.astype(jnp.float32)
    cc = jnp.sum(oh, axis=0, keepdims=True)
    pps = []
    for c in range(NC):
        h = jnp.maximum(dv * (s_ref[c] + tp_ref[c]) + b_ref[c], 0.0)
        pps.append(lax.dot_general(oh, h, (((0,), (0,)), ((), ())),
                                   preferred_element_type=jnp.float32))

    @pl.when(i == 0)
    def _():
        pool_acc[0] = pps[0]
        pool_acc[1] = pps[1]
        cnt_acc[...] = cc

    @pl.when(i > 0)
    def _():
        pool_acc[0] += pps[0]
        pool_acc[1] += pps[1]
        cnt_acc[...] += cc

    @pl.when(i == n_i - 1)
    def _():
        halves = []
        for c in range(NC):
            pc = jnp.zeros((B, HALF), jnp.float32)
            for k in range(4):
                pick = jnp.dot(rowsel_ref[k], pool_acc[c],
                               preferred_element_type=jnp.float32)
                pc = pc + pick[:, k * HALF:(k + 1) * HALF]
            halves.append(pc)
        pooled = jnp.concatenate(halves, axis=1)
        cnt = lax.dot_general(z_ref[...], cnt_acc[...], (((1,), (1,)), ((), ())),
                              preferred_element_type=jnp.float32)
        scl = 1.0 / (jnp.maximum(cnt, 1.0) * jnp.sqrt(cnt + 1e-6))
        o_ref[...] = jnp.dot(pooled * scl, wout_ref[...],
                             preferred_element_type=jnp.float32) + bout_ref[...]


def _tc_final(s_pack, tp, dvp, b_pack, batch_pack, rowsel, z, wout, bout):
    return pl.pallas_call(
        _tc_final_body,
        grid=(NP_ACC // R_P,),
        in_specs=[
            pl.BlockSpec((NC, R_P, 128), lambda i: (0, i, 0)),
            pl.BlockSpec((NC, R_P, 128), lambda i: (0, i, 0)),
            pl.BlockSpec((R_P, 128), lambda i: (i, 0)),
            pl.BlockSpec((NC, 1, 128), lambda i: (0, 0, 0)),
            pl.BlockSpec((R_P, 4), lambda i: (i, 0)),
            pl.BlockSpec((4, B, 4 * B), lambda i: (0, 0, 0)),
            pl.BlockSpec((B, 4 * B), lambda i: (0, 0)),
            pl.BlockSpec((DH, DOUT), lambda i: (0, 0)),
            pl.BlockSpec((1, DOUT), lambda i: (0, 0)),
        ],
        out_specs=pl.BlockSpec((B, DOUT), lambda i: (0, 0)),
        out_shape=jax.ShapeDtypeStruct((B, DOUT), jnp.float32),
        scratch_shapes=[pltpu.VMEM((NC, 4 * B, 128), jnp.float32),
                        pltpu.VMEM((1, 4 * B), jnp.float32)],
    )(s_pack, tp, dvp, b_pack, batch_pack, rowsel, z, wout, bout)


def kernel(x, edge_index, batch, W0, b0, W1, b1, W2, b2, W3, b3, Wout, bout):
    src = edge_index[0]
    dst = edge_index[1]
    pad_i = jnp.arange(PAD, dtype=jnp.int32)
    src_p = jnp.concatenate([src, pad_i % np.int32(N)])
    dst_p = jnp.concatenate([dst, N + (pad_i % np.int32(16))])

    dst_deg = dst_p.reshape(NT, W_DEG, WIN)
    src_t = src_p.reshape(1, NS, W_SP, WIN)
    src_sp = jnp.concatenate([src_t, src_t + N_ACC], axis=0).reshape(NT, W_SP, WIN)
    dst_sp = dst_p.reshape(NS, W_SP, WIN)

    deg_raw = _sc_deg(dst_deg)
    psum = deg_raw[:N_ACC] + deg_raw[N_ACC:]
    pe = jnp.broadcast_to(psum.reshape(NP_ACC, 4, 1),
                          (NP_ACC, 4, HALF)).reshape(NP_ACC, 128)
    x_pack = x.reshape(NP, 4 * DIN)

    i4 = jnp.eye(4, dtype=jnp.float32)
    m0 = jnp.concatenate([jnp.kron(i4, W0[:, :HALF]),
                          jnp.kron(i4, W0[:, HALF:])], axis=1)
    dvp, t = _tc_prep(pe, x_pack, m0)
    for (w, b) in ((W1, b0), (W2, b1), (W3, b2)):
        mw = jnp.concatenate([
            jnp.concatenate([jnp.kron(i4, w[:HALF, :HALF]),
                             jnp.kron(i4, w[:HALF, HALF:])], axis=1),
            jnp.concatenate([jnp.kron(i4, w[HALF:, :HALF]),
                             jnp.kron(i4, w[HALF:, HALF:])], axis=1),
        ], axis=0)
        bp = jnp.tile(b.reshape(NC, 1, HALF), (1, 1, 4))
        s = _sc_spmm(t.reshape(NC * N_ACC, HALF), src_sp, dst_sp)
        t = _tc_layer(s.reshape(NC, NP_ACC, 128), t, dvp, mw, bp)
    s = _sc_spmm(t.reshape(NC * N_ACC, HALF), src_sp, dst_sp)
    batch_pack = jnp.concatenate(
        [batch.astype(jnp.int32).reshape(NP, 4),
         jnp.full((NP_ACC - NP, 4), -1, jnp.int32)])
    eye_b = jnp.eye(B, dtype=jnp.float32)
    rowsel = jnp.stack([jnp.kron(eye_b, jnp.eye(4, dtype=jnp.float32)[k:k + 1])
                        for k in range(4)])
    z = jnp.kron(eye_b, jnp.ones((1, 4), jnp.float32))
    return _tc_final(s.reshape(NC, NP_ACC, 128), t, dvp,
                     jnp.tile(b3.reshape(NC, 1, HALF), (1, 1, 4)), batch_pack,
                     rowsel, z, Wout, bout.reshape(1, DOUT))
